# baseline (speedup 1.0000x reference)
import functools

import jax
import jax.numpy as jnp
from jax import lax
from jax.experimental import pallas as pl
from jax.experimental.pallas import tpu as pltpu
from jax.experimental.pallas import tpu_sc as plsc

N = 10000
E = 320000
D_IN = 128
D_H = 128
D_OUT = 64

NC = 2
NS = 16
NW = NC * NS

NP = 10240
RPS = NP // NS
CH = 128
NB = 2
EW = ((E // NW + NB * CH - 1) // (NB * CH)) * (NB * CH)
EPAD = NW * EW
NCHUNK = EW // CH
NG = NCHUNK // NB

R = 1280
GRID = NP // R


def _make_sc_agg(D, with_deg):
  mesh = plsc.VectorSubcoreMesh(
      core_axis_name="c", subcore_axis_name="s", num_cores=NC, num_subcores=NS)

  out_type = jax.ShapeDtypeStruct((NC, NP, D), jnp.float32)
  if with_deg:
    out_type = [out_type, jax.ShapeDtypeStruct((NC, NP), jnp.float32)]

  scratch = [
      pltpu.VMEM((NB, 2, CH), jnp.int32),
      pltpu.VMEM((NB, CH, D), jnp.float32),
      pltpu.VMEM_SHARED((NP, D), jnp.float32),
  ] + [pltpu.SemaphoreType.DMA] * (2 * NB)
  if with_deg:
    scratch += [
        pltpu.VMEM((CH,), jnp.float32),
        pltpu.VMEM_SHARED((NP,), jnp.float32),
    ]

  def body(h_hbm, e_hbm, z2_hbm, *rest):
    if with_deg:
      (z1_hbm, out_hbm, deg_hbm, ibuf, rows_v, acc_sh, *tail) = rest
      isems = tail[:NB]
      gsems = tail[NB:2 * NB]
      ones_v, deg_sh = tail[2 * NB:]
    else:
      out_hbm, ibuf, rows_v, acc_sh, *tail = rest
      isems = tail[:NB]
      gsems = tail[NB:]

    c = lax.axis_index("c")
    s = lax.axis_index("s")
    w = s * NC + c

    pltpu.sync_copy(z2_hbm.at[pl.ds(s * RPS, RPS)],
                    acc_sh.at[pl.ds(s * RPS, RPS)])
    if with_deg:
      pltpu.sync_copy(z1_hbm.at[pl.ds(s * RPS, RPS)],
                      deg_sh.at[pl.ds(s * RPS, RPS)])
      for i in range(CH // 16):
        ones_v[pl.ds(i * 16, 16)] = jnp.ones((16,), jnp.float32)
    plsc.subcore_barrier()

    def start_idx(j, b):
      pltpu.async_copy(e_hbm.at[w, j], ibuf.at[b], isems[b])

    def wait_idx_start_gather(j, b):
      pltpu.make_async_copy(e_hbm.at[w, j], ibuf.at[b], isems[b]).wait()
      pltpu.async_copy(h_hbm.at[ibuf.at[b, 0]], rows_v.at[b], gsems[b])

    def drain_and_scatter(j, b):
      pltpu.make_async_copy(h_hbm.at[ibuf.at[b, 0]], rows_v.at[b],
                            gsems[b]).wait()
      pltpu.sync_copy(rows_v.at[b], acc_sh.at[ibuf.at[b, 1]], add=True)
      if with_deg:
        pltpu.sync_copy(ones_v, deg_sh.at[ibuf.at[b, 1]], add=True)

    for b in range(NB):
      start_idx(b, b)
    for b in range(NB):
      wait_idx_start_gather(b, b)

    def group(g, carry):
      for b in range(NB):
        j = g * NB + b
        drain_and_scatter(j, b)
        start_idx(j + NB, b)
        wait_idx_start_gather(j + NB, b)
      return carry

    lax.fori_loop(0, NG - 1, group, 0)
    for b in range(NB):
      drain_and_scatter((NG - 1) * NB + b, b)
    plsc.subcore_barrier()

    pltpu.sync_copy(acc_sh.at[pl.ds(s * RPS, RPS)],
                    out_hbm.at[c, pl.ds(s * RPS, RPS)])
    if with_deg:
      pltpu.sync_copy(deg_sh.at[pl.ds(s * RPS, RPS)],
                      deg_hbm.at[c, pl.ds(s * RPS, RPS)])

  params = None
  if D % 128 != 0:
    params = pltpu.CompilerParams(use_tc_tiling_on_sc=False)
  return pl.kernel(body, out_type=out_type, mesh=mesh, scratch_types=scratch,
                   compiler_params=params,
                   name=f"sc_agg_d{D}" + ("_deg" if with_deg else ""))


_sc_agg_deg = _make_sc_agg(D_H, True)
_sc_agg = _make_sc_agg(D_H, False)
_sc_agg64 = _make_sc_agg(D_OUT, False)


def _dot(a, b):
  return jnp.dot(a, b, preferred_element_type=jnp.float32)


def _tc_layer0_body(h_ref, p_ref, d_ref, ws_ref, wn_ref, b_ref,
                    o_ref, invd_ref):
  invd = 1.0 / jnp.maximum(d_ref[0] + d_ref[1], 1.0)
  invd_ref[...] = invd
  agg = (p_ref[0] + p_ref[1]) * invd
  y = _dot(h_ref[...], ws_ref[...]) + _dot(agg, wn_ref[...]) + b_ref[...]
  o_ref[...] = jnp.maximum(y, 0.0)


def _tc_layer1_body(h_ref, p_ref, invd_ref, ws_ref, wn_ref, b_ref, wn2_ref,
                    o_ref, z_ref):
  agg = (p_ref[0] + p_ref[1]) * invd_ref[...]
  y = _dot(h_ref[...], ws_ref[...]) + _dot(agg, wn_ref[...]) + b_ref[...]
  h2 = jnp.maximum(y, 0.0)
  o_ref[...] = h2
  z_ref[...] = _dot(h2, wn2_ref[...])


def _tc_final_body(h_ref, p_ref, invd_ref, ws_ref, b_ref, o_ref):
  agg = (p_ref[0] + p_ref[1]) * invd_ref[...]
  o_ref[...] = _dot(h_ref[...], ws_ref[...]) + agg + b_ref[...]


def _row_block(d):
  return pl.BlockSpec((R, d), lambda i: (i, 0))


def _part_block(d):
  return pl.BlockSpec((NC, R, d), lambda i: (0, i, 0))


def _full(shape):
  return pl.BlockSpec(shape, lambda i: tuple(0 for _ in shape))


_tc_layer0 = pl.pallas_call(
    _tc_layer0_body,
    grid=(GRID,),
    in_specs=[_row_block(D_H), _part_block(D_H), _part_block(1),
              _full((D_IN, D_H)), _full((D_IN, D_H)), _full((1, D_H))],
    out_specs=[_row_block(D_H), _row_block(1)],
    out_shape=[jax.ShapeDtypeStruct((NP, D_H), jnp.float32),
               jax.ShapeDtypeStruct((NP, 1), jnp.float32)],
)

_tc_layer1 = pl.pallas_call(
    _tc_layer1_body,
    grid=(GRID,),
    in_specs=[_row_block(D_H), _part_block(D_H), _row_block(1),
              _full((D_H, D_H)), _full((D_H, D_H)), _full((1, D_H)),
              _full((D_H, D_OUT))],
    out_specs=[_row_block(D_H), _row_block(D_OUT)],
    out_shape=[jax.ShapeDtypeStruct((NP, D_H), jnp.float32),
               jax.ShapeDtypeStruct((NP, D_OUT), jnp.float32)],
)

_tc_final = pl.pallas_call(
    _tc_final_body,
    grid=(GRID,),
    in_specs=[_row_block(D_H), _part_block(D_OUT), _row_block(1),
              _full((D_H, D_OUT)), _full((1, D_OUT))],
    out_specs=_row_block(D_OUT),
    out_shape=jax.ShapeDtypeStruct((NP, D_OUT), jnp.float32),
)


def kernel(x, edge_index, W_self0, W_neigh0, b0, W_self1, W_neigh1, b1,
           W_self2, W_neigh2, b2):
  src = edge_index[0]
  dst = edge_index[1]
  src_p = jnp.concatenate(
      [src, jnp.zeros((EPAD - E,), jnp.int32)]).reshape(NW, NCHUNK, CH)
  dst_p = jnp.concatenate(
      [dst, jnp.full((EPAD - E,), N, jnp.int32)]).reshape(NW, NCHUNK, CH)
  e_pk = jnp.stack([src_p, dst_p], axis=2)

  h0 = jnp.pad(x, ((0, NP - N), (0, 0)))
  z2d = jnp.zeros((NP, D_H), jnp.float32)
  z1d = jnp.zeros((NP,), jnp.float32)

  p0, degp = _sc_agg_deg(h0, e_pk, z2d, z1d)
  h1, invd = _tc_layer0(h0, p0, degp[..., None], W_self0, W_neigh0,
                        b0.reshape(1, D_H))
  p1 = _sc_agg(h1, e_pk, z2d)
  h2, z2 = _tc_layer1(h1, p1, invd, W_self1, W_neigh1, b1.reshape(1, D_H),
                      W_neigh2)
  pz = _sc_agg64(z2, e_pk, z2d[:, :D_OUT])
  out = _tc_final(h2, pz, invd, W_self2, b2.reshape(1, D_OUT))
  return out[:N]


# speedup vs baseline: 1.1726x; 1.1726x over previous
import functools

import jax
import jax.numpy as jnp
from jax import lax
from jax.experimental import pallas as pl
from jax.experimental.pallas import tpu as pltpu
from jax.experimental.pallas import tpu_sc as plsc

N = 10000
E = 320000
D_IN = 128
D_H = 128
D_OUT = 64

NC = 2
NS = 16
NW = NC * NS

NP = 10240
RPS = NP // NS
CH = 128
NB = 2
EW = ((E // NW + NB * CH - 1) // (NB * CH)) * (NB * CH)
EPAD = NW * EW
NCHUNK = EW // CH
NG = NCHUNK // NB

R = 1280
GRID = NP // R


def _make_sc_agg(D, with_deg):
  mesh = plsc.VectorSubcoreMesh(
      core_axis_name="c", subcore_axis_name="s", num_cores=NC, num_subcores=NS)

  out_type = jax.ShapeDtypeStruct((NC, NP, D), jnp.float32)
  if with_deg:
    out_type = [out_type, jax.ShapeDtypeStruct((NC, NP), jnp.float32)]

  IB = 4

  scratch = [
      pltpu.VMEM((IB, CH), jnp.int32),
      pltpu.VMEM((IB, CH), jnp.int32),
      pltpu.VMEM((NB, CH, D), jnp.float32),
      pltpu.VMEM_SHARED((NP, D), jnp.float32),
  ] + [pltpu.SemaphoreType.DMA] * (2 * IB + NB)
  if with_deg:
    scratch += [
        pltpu.VMEM((CH,), jnp.float32),
        pltpu.VMEM_SHARED((NP,), jnp.float32),
    ]

  def body(h_hbm, src_hbm, dst_hbm, z2_hbm, *rest):
    if with_deg:
      (z1_hbm, out_hbm, deg_hbm, sbuf, dbuf, rows_v, acc_sh, *tail) = rest
    else:
      (out_hbm, sbuf, dbuf, rows_v, acc_sh, *tail) = rest
    ssems = tail[:IB]
    dsems = tail[IB:2 * IB]
    gsems = tail[2 * IB:2 * IB + NB]
    if with_deg:
      ones_v, deg_sh = tail[2 * IB + NB:]

    c = lax.axis_index("c")
    s = lax.axis_index("s")
    w = s * NC + c

    pltpu.sync_copy(z2_hbm.at[pl.ds(s * RPS, RPS)],
                    acc_sh.at[pl.ds(s * RPS, RPS)])
    if with_deg:
      pltpu.sync_copy(z1_hbm.at[pl.ds(s * RPS, RPS)],
                      deg_sh.at[pl.ds(s * RPS, RPS)])
      for i in range(CH // 16):
        ones_v[pl.ds(i * 16, 16)] = jnp.ones((16,), jnp.float32)
    plsc.subcore_barrier()

    def start_idx(j, q):
      pltpu.async_copy(src_hbm.at[w, j], sbuf.at[q], ssems[q])
      pltpu.async_copy(dst_hbm.at[w, j], dbuf.at[q], dsems[q])

    def wait_idx(j, q):
      pltpu.make_async_copy(src_hbm.at[w, j], sbuf.at[q], ssems[q]).wait()
      pltpu.make_async_copy(dst_hbm.at[w, j], dbuf.at[q], dsems[q]).wait()

    def start_gather(q, b):
      pltpu.async_copy(h_hbm.at[sbuf.at[q]], rows_v.at[b], gsems[b])

    def wait_gather(q, b):
      pltpu.make_async_copy(h_hbm.at[sbuf.at[q]], rows_v.at[b],
                            gsems[b]).wait()

    def scatter(q, b):
      pltpu.sync_copy(rows_v.at[b], acc_sh.at[dbuf.at[q]], add=True)
      if with_deg:
        pltpu.sync_copy(ones_v, deg_sh.at[dbuf.at[q]], add=True)

    for q in range(IB):
      start_idx(q, q)
    for j in range(NB):
      wait_idx(j, j)
      start_gather(j, j)

    def group(g, carry):
      for u in range(IB):
        j = g * IB + u
        q = u
        b = u % NB
        wait_gather(q, b)
        scatter(q, b)

        @pl.when(j + IB < NCHUNK)
        def _():
          start_idx(j + IB, q)

        @pl.when(j + NB < NCHUNK)
        def _():
          wait_idx(j + NB, (u + NB) % IB)
          start_gather((u + NB) % IB, b)
      return carry

    lax.fori_loop(0, NCHUNK // IB, group, 0)
    plsc.subcore_barrier()

    pltpu.sync_copy(acc_sh.at[pl.ds(s * RPS, RPS)],
                    out_hbm.at[c, pl.ds(s * RPS, RPS)])
    if with_deg:
      pltpu.sync_copy(deg_sh.at[pl.ds(s * RPS, RPS)],
                      deg_hbm.at[c, pl.ds(s * RPS, RPS)])

  params = None
  if D % 128 != 0:
    params = pltpu.CompilerParams(use_tc_tiling_on_sc=False)
  return pl.kernel(body, out_type=out_type, mesh=mesh, scratch_types=scratch,
                   compiler_params=params,
                   name=f"sc_agg_d{D}" + ("_deg" if with_deg else ""))


_sc_agg_deg = _make_sc_agg(D_H, True)
_sc_agg = _make_sc_agg(D_H, False)
_sc_agg64 = _make_sc_agg(D_OUT, False)


def _dot(a, b):
  return jnp.dot(a, b, preferred_element_type=jnp.float32)


def _tc_layer0_body(h_ref, p_ref, d_ref, ws_ref, wn_ref, b_ref,
                    o_ref, invd_ref):
  invd = 1.0 / jnp.maximum(d_ref[0] + d_ref[1], 1.0)
  invd_ref[...] = invd
  agg = (p_ref[0] + p_ref[1]) * invd
  y = _dot(h_ref[...], ws_ref[...]) + _dot(agg, wn_ref[...]) + b_ref[...]
  o_ref[...] = jnp.maximum(y, 0.0)


def _tc_layer1_body(h_ref, p_ref, invd_ref, ws_ref, wn_ref, b_ref, wn2_ref,
                    o_ref, z_ref):
  agg = (p_ref[0] + p_ref[1]) * invd_ref[...]
  y = _dot(h_ref[...], ws_ref[...]) + _dot(agg, wn_ref[...]) + b_ref[...]
  h2 = jnp.maximum(y, 0.0)
  o_ref[...] = h2
  z_ref[...] = _dot(h2, wn2_ref[...])


def _tc_final_body(h_ref, p_ref, invd_ref, ws_ref, b_ref, o_ref):
  agg = (p_ref[0] + p_ref[1]) * invd_ref[...]
  o_ref[...] = _dot(h_ref[...], ws_ref[...]) + agg + b_ref[...]


def _row_block(d):
  return pl.BlockSpec((R, d), lambda i: (i, 0))


def _part_block(d):
  return pl.BlockSpec((NC, R, d), lambda i: (0, i, 0))


def _full(shape):
  return pl.BlockSpec(shape, lambda i: tuple(0 for _ in shape))


_tc_layer0 = pl.pallas_call(
    _tc_layer0_body,
    grid=(GRID,),
    in_specs=[_row_block(D_H), _part_block(D_H), _part_block(1),
              _full((D_IN, D_H)), _full((D_IN, D_H)), _full((1, D_H))],
    out_specs=[_row_block(D_H), _row_block(1)],
    out_shape=[jax.ShapeDtypeStruct((NP, D_H), jnp.float32),
               jax.ShapeDtypeStruct((NP, 1), jnp.float32)],
)

_tc_layer1 = pl.pallas_call(
    _tc_layer1_body,
    grid=(GRID,),
    in_specs=[_row_block(D_H), _part_block(D_H), _row_block(1),
              _full((D_H, D_H)), _full((D_H, D_H)), _full((1, D_H)),
              _full((D_H, D_OUT))],
    out_specs=[_row_block(D_H), _row_block(D_OUT)],
    out_shape=[jax.ShapeDtypeStruct((NP, D_H), jnp.float32),
               jax.ShapeDtypeStruct((NP, D_OUT), jnp.float32)],
)

_tc_final = pl.pallas_call(
    _tc_final_body,
    grid=(GRID,),
    in_specs=[_row_block(D_H), _part_block(D_OUT), _row_block(1),
              _full((D_H, D_OUT)), _full((1, D_OUT))],
    out_specs=_row_block(D_OUT),
    out_shape=jax.ShapeDtypeStruct((NP, D_OUT), jnp.float32),
)


def kernel(x, edge_index, W_self0, W_neigh0, b0, W_self1, W_neigh1, b1,
           W_self2, W_neigh2, b2):
  src = edge_index[0]
  dst = edge_index[1]
  src_p = jnp.concatenate(
      [src, jnp.zeros((EPAD - E,), jnp.int32)]).reshape(NW, NCHUNK, CH)
  dst_p = jnp.concatenate(
      [dst, jnp.full((EPAD - E,), N, jnp.int32)]).reshape(NW, NCHUNK, CH)

  h0 = jnp.pad(x, ((0, NP - N), (0, 0)))
  z2d = jnp.zeros((NP, D_H), jnp.float32)
  z1d = jnp.zeros((NP,), jnp.float32)

  p0, degp = _sc_agg_deg(h0, src_p, dst_p, z2d, z1d)
  h1, invd = _tc_layer0(h0, p0, degp[..., None], W_self0, W_neigh0,
                        b0.reshape(1, D_H))
  p1 = _sc_agg(h1, src_p, dst_p, z2d)
  h2, z2 = _tc_layer1(h1, p1, invd, W_self1, W_neigh1, b1.reshape(1, D_H),
                      W_neigh2)
  pz = _sc_agg64(z2, src_p, dst_p, z2d[:, :D_OUT])
  out = _tc_final(h2, pz, invd, W_self2, b2.reshape(1, D_OUT))
  return out[:N]


# speedup vs baseline: 2.4362x; 2.0776x over previous
import jax
import jax.numpy as jnp
from jax import lax
from jax.experimental import pallas as pl
from jax.experimental.pallas import tpu as pltpu
from jax.experimental.pallas import tpu_sc as plsc

N = 10000
E = 320000
D_IN = 128
D_H = 128
D_OUT = 64
DC = 64

NC = 2
NS = 16
NW = NC * NS

NP = 10240
RPS = NP // NS
CH = 128
NB = 2
EW = ((E // NW + NB * CH - 1) // (NB * CH)) * (NB * CH)
EPAD = NW * EW
NCHUNK = EW // CH

R = 1280
GRID = NP // R


def _make_sc_agg(with_deg):
  mesh = plsc.VectorSubcoreMesh(
      core_axis_name="c", subcore_axis_name="s", num_cores=NC, num_subcores=NS)

  out_type = jax.ShapeDtypeStruct((NC, NP, DC), jnp.float32)
  if with_deg:
    out_type = [out_type, jax.ShapeDtypeStruct((NC, NP), jnp.float32)]

  IB = 4

  scratch = [
      pltpu.VMEM((IB, CH), jnp.int32),
      pltpu.VMEM((IB, CH), jnp.int32),
      pltpu.VMEM((NB, CH, DC), jnp.float32),
      pltpu.VMEM_SHARED((NP, DC), jnp.float32),
      pltpu.VMEM_SHARED((NP, DC), jnp.float32),
  ] + [pltpu.SemaphoreType.DMA] * (2 * IB + NB)
  if with_deg:
    scratch += [
        pltpu.VMEM((CH,), jnp.float32),
        pltpu.VMEM_SHARED((NP,), jnp.float32),
    ]

  def body(h_hbm, src_hbm, dst_hbm, z2_hbm, *rest):
    if with_deg:
      (z1_hbm, out_hbm, deg_hbm, sbuf, dbuf, rows_v, h_sh, acc_sh,
       *tail) = rest
    else:
      (out_hbm, sbuf, dbuf, rows_v, h_sh, acc_sh, *tail) = rest
    ssems = tail[:IB]
    dsems = tail[IB:2 * IB]
    gsems = tail[2 * IB:2 * IB + NB]
    if with_deg:
      ones_v, deg_sh = tail[2 * IB + NB:]

    c = lax.axis_index("c")
    s = lax.axis_index("s")
    w = s * NC + c

    pltpu.sync_copy(h_hbm.at[pl.ds(s * RPS, RPS)],
                    h_sh.at[pl.ds(s * RPS, RPS)])
    pltpu.sync_copy(z2_hbm.at[pl.ds(s * RPS, RPS)],
                    acc_sh.at[pl.ds(s * RPS, RPS)])
    if with_deg:
      pltpu.sync_copy(z1_hbm.at[pl.ds(s * RPS, RPS)],
                      deg_sh.at[pl.ds(s * RPS, RPS)])
      for i in range(CH // 16):
        ones_v[pl.ds(i * 16, 16)] = jnp.ones((16,), jnp.float32)
    plsc.subcore_barrier()

    def start_idx(j, q):
      pltpu.async_copy(src_hbm.at[w, j], sbuf.at[q], ssems[q])
      pltpu.async_copy(dst_hbm.at[w, j], dbuf.at[q], dsems[q])

    def wait_idx(j, q):
      pltpu.make_async_copy(src_hbm.at[w, j], sbuf.at[q], ssems[q]).wait()
      pltpu.make_async_copy(dst_hbm.at[w, j], dbuf.at[q], dsems[q]).wait()

    def start_gather(q, b):
      pltpu.async_copy(h_sh.at[sbuf.at[q]], rows_v.at[b], gsems[b])

    def wait_gather(q, b):
      pltpu.make_async_copy(h_sh.at[sbuf.at[q]], rows_v.at[b],
                            gsems[b]).wait()

    def scatter(q, b):
      pltpu.sync_copy(rows_v.at[b], acc_sh.at[dbuf.at[q]], add=True)
      if with_deg:
        pltpu.sync_copy(ones_v, deg_sh.at[dbuf.at[q]], add=True)

    for q in range(IB):
      start_idx(q, q)
    for j in range(NB):
      wait_idx(j, j)
      start_gather(j, j)

    def group(g, carry):
      for u in range(IB):
        j = g * IB + u
        q = u
        b = u % NB
        wait_gather(q, b)
        scatter(q, b)

        @pl.when(j + IB < NCHUNK)
        def _():
          start_idx(j + IB, q)

        @pl.when(j + NB < NCHUNK)
        def _():
          wait_idx(j + NB, (u + NB) % IB)
          start_gather((u + NB) % IB, b)
      return carry

    lax.fori_loop(0, NCHUNK // IB, group, 0)
    plsc.subcore_barrier()

    pltpu.sync_copy(acc_sh.at[pl.ds(s * RPS, RPS)],
                    out_hbm.at[c, pl.ds(s * RPS, RPS)])
    if with_deg:
      pltpu.sync_copy(deg_sh.at[pl.ds(s * RPS, RPS)],
                      deg_hbm.at[c, pl.ds(s * RPS, RPS)])

  params = pltpu.CompilerParams(use_tc_tiling_on_sc=False)
  return pl.kernel(body, out_type=out_type, mesh=mesh, scratch_types=scratch,
                   compiler_params=params,
                   name="sc_agg" + ("_deg" if with_deg else ""))


_sc_agg_deg = _make_sc_agg(True)
_sc_agg = _make_sc_agg(False)


def _dot(a, b):
  return jnp.dot(a, b, preferred_element_type=jnp.float32)


def _tc_layer0_body(h_ref, pa_ref, pb_ref, d_ref, ws_ref, wn_ref, b_ref,
                    o_ref, invd_ref):
  invd = 1.0 / jnp.maximum(d_ref[0] + d_ref[1], 1.0)
  invd_ref[...] = invd
  agg = jnp.concatenate(
      [(pa_ref[0] + pa_ref[1]) * invd, (pb_ref[0] + pb_ref[1]) * invd],
      axis=1)
  y = _dot(h_ref[...], ws_ref[...]) + _dot(agg, wn_ref[...]) + b_ref[...]
  h1 = jnp.maximum(y, 0.0)
  o_ref[0] = h1[:, :DC]
  o_ref[1] = h1[:, DC:]


def _tc_layer1_body(h_ref, pa_ref, pb_ref, invd_ref, ws_ref, wn_ref, b_ref,
                    wn2_ref, o_ref, z_ref):
  agg = jnp.concatenate(
      [(pa_ref[0] + pa_ref[1]) * invd_ref[...],
       (pb_ref[0] + pb_ref[1]) * invd_ref[...]], axis=1)
  h = jnp.concatenate([h_ref[0], h_ref[1]], axis=1)
  y = _dot(h, ws_ref[...]) + _dot(agg, wn_ref[...]) + b_ref[...]
  h2 = jnp.maximum(y, 0.0)
  o_ref[...] = h2
  z_ref[...] = _dot(h2, wn2_ref[...])


def _tc_final_body(h_ref, p_ref, invd_ref, ws_ref, b_ref, o_ref):
  agg = (p_ref[0] + p_ref[1]) * invd_ref[...]
  o_ref[...] = _dot(h_ref[...], ws_ref[...]) + agg + b_ref[...]


def _row_block(d):
  return pl.BlockSpec((R, d), lambda i: (i, 0))


def _half_block():
  return pl.BlockSpec((2, R, DC), lambda i: (0, i, 0))


def _part_block(d):
  return pl.BlockSpec((NC, R, d), lambda i: (0, i, 0))


def _full(shape):
  return pl.BlockSpec(shape, lambda i: tuple(0 for _ in shape))


_tc_layer0 = pl.pallas_call(
    _tc_layer0_body,
    grid=(GRID,),
    in_specs=[_row_block(D_H), _part_block(DC), _part_block(DC),
              _part_block(1),
              _full((D_IN, D_H)), _full((D_IN, D_H)), _full((1, D_H))],
    out_specs=[_half_block(), _row_block(1)],
    out_shape=[jax.ShapeDtypeStruct((2, NP, DC), jnp.float32),
               jax.ShapeDtypeStruct((NP, 1), jnp.float32)],
)

_tc_layer1 = pl.pallas_call(
    _tc_layer1_body,
    grid=(GRID,),
    in_specs=[_half_block(), _part_block(DC), _part_block(DC), _row_block(1),
              _full((D_H, D_H)), _full((D_H, D_H)), _full((1, D_H)),
              _full((D_H, D_OUT))],
    out_specs=[_row_block(D_H), _row_block(D_OUT)],
    out_shape=[jax.ShapeDtypeStruct((NP, D_H), jnp.float32),
               jax.ShapeDtypeStruct((NP, D_OUT), jnp.float32)],
)

_tc_final = pl.pallas_call(
    _tc_final_body,
    grid=(GRID,),
    in_specs=[_row_block(D_H), _part_block(D_OUT), _row_block(1),
              _full((D_H, D_OUT)), _full((1, D_OUT))],
    out_specs=_row_block(D_OUT),
    out_shape=jax.ShapeDtypeStruct((NP, D_OUT), jnp.float32),
)


def kernel(x, edge_index, W_self0, W_neigh0, b0, W_self1, W_neigh1, b1,
           W_self2, W_neigh2, b2):
  src = edge_index[0]
  dst = edge_index[1]
  src_p = jnp.concatenate(
      [src, jnp.zeros((EPAD - E,), jnp.int32)]).reshape(NW, NCHUNK, CH)
  dst_p = jnp.concatenate(
      [dst, jnp.full((EPAD - E,), N, jnp.int32)]).reshape(NW, NCHUNK, CH)

  h0 = jnp.pad(x, ((0, NP - N), (0, 0)))
  h0a = h0[:, :DC]
  h0b = h0[:, DC:]
  z2d = jnp.zeros((NP, DC), jnp.float32)
  z1d = jnp.zeros((NP,), jnp.float32)

  pa0, degp = _sc_agg_deg(h0a, src_p, dst_p, z2d, z1d)
  pb0 = _sc_agg(h0b, src_p, dst_p, z2d)
  h1h, invd = _tc_layer0(h0, pa0, pb0, degp[..., None], W_self0, W_neigh0,
                         b0.reshape(1, D_H))
  pa1 = _sc_agg(h1h[0], src_p, dst_p, z2d)
  pb1 = _sc_agg(h1h[1], src_p, dst_p, z2d)
  h2, z2 = _tc_layer1(h1h, pa1, pb1, invd, W_self1, W_neigh1,
                      b1.reshape(1, D_H), W_neigh2)
  pz = _sc_agg(z2, src_p, dst_p, z2d)
  out = _tc_final(h2, pz, invd, W_self2, b2.reshape(1, D_OUT))
  return out[:N]


# speedup vs baseline: 2.5100x; 1.0303x over previous
import jax
import jax.numpy as jnp
from jax import lax
from jax.experimental import pallas as pl
from jax.experimental.pallas import tpu as pltpu
from jax.experimental.pallas import tpu_sc as plsc

N = 10000
E = 320000
D_IN = 128
D_H = 128
D_OUT = 64
DC = 64

NC = 2
NS = 16
NW = NC * NS

NP = 10240
RPS = NP // NS
CH = 128
NB = 2
IB = 4

EW1 = ((E // NS + IB * CH - 1) // (IB * CH)) * (IB * CH)
NCHUNK1 = EW1 // CH
EW2 = EW1 // 2
NCHUNK2 = EW2 // CH
EPAD = NS * EW1

R = 1280
GRID = NP // R


def _sc_scratch(with_deg):
  scratch = [
      pltpu.VMEM((IB, CH), jnp.int32),
      pltpu.VMEM((IB, CH), jnp.int32),
      pltpu.VMEM((NB, CH, DC), jnp.float32),
      pltpu.VMEM_SHARED((NP, DC), jnp.float32),
      pltpu.VMEM_SHARED((NP, DC), jnp.float32),
  ] + [pltpu.SemaphoreType.DMA] * (2 * IB + NB)
  if with_deg:
    scratch += [
        pltpu.VMEM((CH,), jnp.float32),
        pltpu.VMEM_SHARED((NP,), jnp.float32),
    ]
  return scratch


def _edge_loop(nchunk, src_hbm, dst_hbm, w, sbuf, dbuf, rows_v, h_sh, acc_sh,
               ssems, dsems, gsems, deg_chunk=None):
  def start_idx(j, q):
    pltpu.async_copy(src_hbm.at[w, j], sbuf.at[q], ssems[q])
    pltpu.async_copy(dst_hbm.at[w, j], dbuf.at[q], dsems[q])

  def wait_idx(j, q):
    pltpu.make_async_copy(src_hbm.at[w, j], sbuf.at[q], ssems[q]).wait()
    pltpu.make_async_copy(dst_hbm.at[w, j], dbuf.at[q], dsems[q]).wait()

  def start_gather(q, b):
    pltpu.async_copy(h_sh.at[sbuf.at[q]], rows_v.at[b], gsems[b])

  def wait_gather(q, b):
    pltpu.make_async_copy(h_sh.at[sbuf.at[q]], rows_v.at[b], gsems[b]).wait()

  for q in range(IB):
    start_idx(q, q)
  for j in range(NB):
    wait_idx(j, j)
    start_gather(j, j)

  def group(g, carry):
    for u in range(IB):
      q = u
      b = u % NB
      j = g * IB + u
      wait_gather(q, b)
      pltpu.sync_copy(rows_v.at[b], acc_sh.at[dbuf.at[q]], add=True)
      if deg_chunk is not None:
        deg_chunk(j, q)

      @pl.when(j + IB < nchunk)
      def _():
        start_idx(j + IB, q)

      @pl.when(j + NB < nchunk)
      def _():
        wait_idx(j + NB, (u + NB) % IB)
        start_gather((u + NB) % IB, b)
    return carry

  lax.fori_loop(0, nchunk // IB, group, 0)


def _make_sc_layer():
  mesh = plsc.VectorSubcoreMesh(
      core_axis_name="c", subcore_axis_name="s", num_cores=NC, num_subcores=NS)

  out_type = [jax.ShapeDtypeStruct((NC, NP, DC), jnp.float32),
              jax.ShapeDtypeStruct((NC, NP), jnp.float32)]

  def body(h_hbm, src_hbm, dst_hbm, z2_hbm, z1_hbm, out_hbm, deg_hbm,
           sbuf, dbuf, rows_v, h_sh, acc_sh, *tail):
    ssems = tail[:IB]
    dsems = tail[IB:2 * IB]
    gsems = tail[2 * IB:2 * IB + NB]
    ones_v, deg_sh = tail[2 * IB + NB:]

    c = lax.axis_index("c")
    s = lax.axis_index("s")

    pltpu.sync_copy(h_hbm.at[c, pl.ds(s * RPS, RPS)],
                    h_sh.at[pl.ds(s * RPS, RPS)])
    pltpu.sync_copy(z2_hbm.at[pl.ds(s * RPS, RPS)],
                    acc_sh.at[pl.ds(s * RPS, RPS)])
    pltpu.sync_copy(z1_hbm.at[pl.ds(s * RPS, RPS)],
                    deg_sh.at[pl.ds(s * RPS, RPS)])
    for i in range(CH // 16):
      ones_v[pl.ds(i * 16, 16)] = jnp.ones((16,), jnp.float32)
    plsc.subcore_barrier()

    def deg_chunk(j, q):
      @pl.when((j < NCHUNK1 // 2) == (c == 0))
      def _():
        pltpu.sync_copy(ones_v, deg_sh.at[dbuf.at[q]], add=True)

    _edge_loop(NCHUNK1, src_hbm, dst_hbm, s, sbuf, dbuf, rows_v, h_sh,
               acc_sh, ssems, dsems, gsems, deg_chunk)
    plsc.subcore_barrier()

    pltpu.sync_copy(acc_sh.at[pl.ds(s * RPS, RPS)],
                    out_hbm.at[c, pl.ds(s * RPS, RPS)])
    pltpu.sync_copy(deg_sh.at[pl.ds(s * RPS, RPS)],
                    deg_hbm.at[c, pl.ds(s * RPS, RPS)])

  return pl.kernel(body, out_type=out_type, mesh=mesh,
                   scratch_types=_sc_scratch(True),
                   compiler_params=pltpu.CompilerParams(
                       use_tc_tiling_on_sc=False),
                   name="sc_layer_deg")


def _make_sc_layer_nodeg():
  mesh = plsc.VectorSubcoreMesh(
      core_axis_name="c", subcore_axis_name="s", num_cores=NC, num_subcores=NS)

  out_type = jax.ShapeDtypeStruct((NC, NP, DC), jnp.float32)

  def body(h_hbm, src_hbm, dst_hbm, z2_hbm, out_hbm,
           sbuf, dbuf, rows_v, h_sh, acc_sh, *tail):
    ssems = tail[:IB]
    dsems = tail[IB:2 * IB]
    gsems = tail[2 * IB:2 * IB + NB]

    c = lax.axis_index("c")
    s = lax.axis_index("s")

    pltpu.sync_copy(h_hbm.at[c, pl.ds(s * RPS, RPS)],
                    h_sh.at[pl.ds(s * RPS, RPS)])
    pltpu.sync_copy(z2_hbm.at[pl.ds(s * RPS, RPS)],
                    acc_sh.at[pl.ds(s * RPS, RPS)])
    plsc.subcore_barrier()

    _edge_loop(NCHUNK1, src_hbm, dst_hbm, s, sbuf, dbuf, rows_v, h_sh,
               acc_sh, ssems, dsems, gsems)
    plsc.subcore_barrier()

    pltpu.sync_copy(acc_sh.at[pl.ds(s * RPS, RPS)],
                    out_hbm.at[c, pl.ds(s * RPS, RPS)])

  return pl.kernel(body, out_type=out_type, mesh=mesh,
                   scratch_types=_sc_scratch(False),
                   compiler_params=pltpu.CompilerParams(
                       use_tc_tiling_on_sc=False),
                   name="sc_layer")


def _make_sc_split():
  mesh = plsc.VectorSubcoreMesh(
      core_axis_name="c", subcore_axis_name="s", num_cores=NC, num_subcores=NS)

  out_type = jax.ShapeDtypeStruct((NC, NP, DC), jnp.float32)

  def body(h_hbm, src_hbm, dst_hbm, z2_hbm, out_hbm,
           sbuf, dbuf, rows_v, h_sh, acc_sh, *tail):
    ssems = tail[:IB]
    dsems = tail[IB:2 * IB]
    gsems = tail[2 * IB:2 * IB + NB]

    c = lax.axis_index("c")
    s = lax.axis_index("s")
    w = s * NC + c

    pltpu.sync_copy(h_hbm.at[pl.ds(s * RPS, RPS)],
                    h_sh.at[pl.ds(s * RPS, RPS)])
    pltpu.sync_copy(z2_hbm.at[pl.ds(s * RPS, RPS)],
                    acc_sh.at[pl.ds(s * RPS, RPS)])
    plsc.subcore_barrier()

    _edge_loop(NCHUNK2, src_hbm, dst_hbm, w, sbuf, dbuf, rows_v, h_sh,
               acc_sh, ssems, dsems, gsems)
    plsc.subcore_barrier()

    pltpu.sync_copy(acc_sh.at[pl.ds(s * RPS, RPS)],
                    out_hbm.at[c, pl.ds(s * RPS, RPS)])

  return pl.kernel(body, out_type=out_type, mesh=mesh,
                   scratch_types=_sc_scratch(False),
                   compiler_params=pltpu.CompilerParams(
                       use_tc_tiling_on_sc=False),
                   name="sc_split")


_sc_layer_deg = _make_sc_layer()
_sc_layer = _make_sc_layer_nodeg()
_sc_split = _make_sc_split()


def _dot(a, b):
  return jnp.dot(a, b, preferred_element_type=jnp.float32)


def _tc_layer0_body(h_ref, a_ref, d_ref, ws_ref, wn_ref, b_ref,
                    o_ref, invd_ref):
  invd = 1.0 / jnp.maximum(d_ref[0] + d_ref[1], 1.0)
  invd_ref[...] = invd
  agg = jnp.concatenate([a_ref[0] * invd, a_ref[1] * invd], axis=1)
  y = _dot(h_ref[...], ws_ref[...]) + _dot(agg, wn_ref[...]) + b_ref[...]
  h1 = jnp.maximum(y, 0.0)
  o_ref[0] = h1[:, :DC]
  o_ref[1] = h1[:, DC:]


def _tc_layer1_body(h_ref, a_ref, invd_ref, ws_ref, wn_ref, b_ref,
                    wn2_ref, o_ref, z_ref):
  agg = jnp.concatenate(
      [a_ref[0] * invd_ref[...], a_ref[1] * invd_ref[...]], axis=1)
  h = jnp.concatenate([h_ref[0], h_ref[1]], axis=1)
  y = _dot(h, ws_ref[...]) + _dot(agg, wn_ref[...]) + b_ref[...]
  h2 = jnp.maximum(y, 0.0)
  o_ref[...] = h2
  z_ref[...] = _dot(h2, wn2_ref[...])


def _tc_final_body(h_ref, p_ref, invd_ref, ws_ref, b_ref, o_ref):
  agg = (p_ref[0] + p_ref[1]) * invd_ref[...]
  o_ref[...] = _dot(h_ref[...], ws_ref[...]) + agg + b_ref[...]


def _row_block(d):
  return pl.BlockSpec((R, d), lambda i: (i, 0))


def _half_block():
  return pl.BlockSpec((2, R, DC), lambda i: (0, i, 0))


def _part_block(d):
  return pl.BlockSpec((NC, R, d), lambda i: (0, i, 0))


def _full(shape):
  return pl.BlockSpec(shape, lambda i: tuple(0 for _ in shape))


_tc_layer0 = pl.pallas_call(
    _tc_layer0_body,
    grid=(GRID,),
    in_specs=[_row_block(D_H), _part_block(DC), _part_block(1),
              _full((D_IN, D_H)), _full((D_IN, D_H)), _full((1, D_H))],
    out_specs=[_half_block(), _row_block(1)],
    out_shape=[jax.ShapeDtypeStruct((2, NP, DC), jnp.float32),
               jax.ShapeDtypeStruct((NP, 1), jnp.float32)],
)

_tc_layer1 = pl.pallas_call(
    _tc_layer1_body,
    grid=(GRID,),
    in_specs=[_half_block(), _part_block(DC), _row_block(1),
              _full((D_H, D_H)), _full((D_H, D_H)), _full((1, D_H)),
              _full((D_H, D_OUT))],
    out_specs=[_row_block(D_H), _row_block(D_OUT)],
    out_shape=[jax.ShapeDtypeStruct((NP, D_H), jnp.float32),
               jax.ShapeDtypeStruct((NP, D_OUT), jnp.float32)],
)

_tc_final = pl.pallas_call(
    _tc_final_body,
    grid=(GRID,),
    in_specs=[_row_block(D_H), _part_block(D_OUT), _row_block(1),
              _full((D_H, D_OUT)), _full((1, D_OUT))],
    out_specs=_row_block(D_OUT),
    out_shape=jax.ShapeDtypeStruct((NP, D_OUT), jnp.float32),
)


def kernel(x, edge_index, W_self0, W_neigh0, b0, W_self1, W_neigh1, b1,
           W_self2, W_neigh2, b2):
  src = edge_index[0]
  dst = edge_index[1]
  src_pad = jnp.concatenate([src, jnp.zeros((EPAD - E,), jnp.int32)])
  dst_pad = jnp.concatenate([dst, jnp.full((EPAD - E,), N, jnp.int32)])
  src1 = src_pad.reshape(NS, NCHUNK1, CH)
  dst1 = dst_pad.reshape(NS, NCHUNK1, CH)
  src2 = src_pad.reshape(NW, NCHUNK2, CH)
  dst2 = dst_pad.reshape(NW, NCHUNK2, CH)

  h0 = jnp.pad(x, ((0, NP - N), (0, 0)))
  h0h = jnp.stack([h0[:, :DC], h0[:, DC:]])
  z2d = jnp.zeros((NP, DC), jnp.float32)
  z1d = jnp.zeros((NP,), jnp.float32)

  a0, degp = _sc_layer_deg(h0h, src1, dst1, z2d, z1d)
  h1h, invd = _tc_layer0(h0, a0, degp[..., None], W_self0, W_neigh0,
                         b0.reshape(1, D_H))
  a1 = _sc_layer(h1h, src1, dst1, z2d)
  h2, z2 = _tc_layer1(h1h, a1, invd, W_self1, W_neigh1,
                      b1.reshape(1, D_H), W_neigh2)
  pz = _sc_split(z2, src2, dst2, z2d)
  out = _tc_final(h2, pz, invd, W_self2, b2.reshape(1, D_OUT))
  return out[:N]


# speedup vs baseline: 2.5238x; 1.0055x over previous
import jax
import jax.numpy as jnp
from jax import lax
from jax.experimental import pallas as pl
from jax.experimental.pallas import tpu as pltpu
from jax.experimental.pallas import tpu_sc as plsc

N = 10000
E = 320000
D_IN = 128
D_H = 128
D_OUT = 64
DC = 64

NC = 2
NS = 16
NW = NC * NS

NP = 10240
RPS = NP // NS
CH = 128
NB = 2
IB = 4

EW1 = ((E // NS + IB * CH - 1) // (IB * CH)) * (IB * CH)
NCHUNK1 = EW1 // CH
NCHUNK2 = NCHUNK1 // 2
EPAD = NS * EW1

R = 1280
GRID = NP // R


def _sc_scratch(with_deg):
  scratch = [
      pltpu.VMEM((IB, CH), jnp.int32),
      pltpu.VMEM((IB, CH), jnp.int32),
      pltpu.VMEM((NB, CH, DC), jnp.float32),
      pltpu.VMEM_SHARED((NP, DC), jnp.float32),
      pltpu.VMEM_SHARED((NP, DC), jnp.float32),
  ] + [pltpu.SemaphoreType.DMA] * (2 * IB + NB)
  if with_deg:
    scratch += [
        pltpu.VMEM((CH,), jnp.float32),
        pltpu.VMEM_SHARED((NP,), jnp.float32),
    ]
  return scratch


def _zero_acc(rows_v, acc_sh, s):
  for i in range(CH):
    for k in range(DC // 16):
      rows_v[0, i, pl.ds(k * 16, 16)] = jnp.zeros((16,), jnp.float32)
  for t in range(RPS // CH):
    pltpu.sync_copy(rows_v.at[0], acc_sh.at[pl.ds(s * RPS + t * CH, CH)])


def _edge_loop(nchunk, src_hbm, dst_hbm, idx_at, sbuf, dbuf, rows_v, h_sh,
               acc_sh, ssems, dsems, gsems, deg_chunk=None):
  def start_idx(j, q):
    r, jj = idx_at(j)
    pltpu.async_copy(src_hbm.at[r, jj], sbuf.at[q], ssems[q])
    pltpu.async_copy(dst_hbm.at[r, jj], dbuf.at[q], dsems[q])

  def wait_idx(j, q):
    r, jj = idx_at(j)
    pltpu.make_async_copy(src_hbm.at[r, jj], sbuf.at[q], ssems[q]).wait()
    pltpu.make_async_copy(dst_hbm.at[r, jj], dbuf.at[q], dsems[q]).wait()

  def start_gather(q, b):
    pltpu.async_copy(h_sh.at[sbuf.at[q]], rows_v.at[b], gsems[b])

  def wait_gather(q, b):
    pltpu.make_async_copy(h_sh.at[sbuf.at[q]], rows_v.at[b], gsems[b]).wait()

  for q in range(IB):
    start_idx(q, q)
  for j in range(NB):
    wait_idx(j, j)
    start_gather(j, j)

  def group(g, carry):
    for u in range(IB):
      q = u
      b = u % NB
      j = g * IB + u
      wait_gather(q, b)
      pltpu.sync_copy(rows_v.at[b], acc_sh.at[dbuf.at[q]], add=True)
      if deg_chunk is not None:
        deg_chunk(j, q)

      @pl.when(j + IB < nchunk)
      def _():
        start_idx(j + IB, q)

      @pl.when(j + NB < nchunk)
      def _():
        wait_idx(j + NB, (u + NB) % IB)
        start_gather((u + NB) % IB, b)
    return carry

  lax.fori_loop(0, nchunk // IB, group, 0)


def _make_sc_layer():
  mesh = plsc.VectorSubcoreMesh(
      core_axis_name="c", subcore_axis_name="s", num_cores=NC, num_subcores=NS)

  out_type = [jax.ShapeDtypeStruct((NC, NP, DC), jnp.float32),
              jax.ShapeDtypeStruct((NC, NP), jnp.float32)]

  def body(h_hbm, src_hbm, dst_hbm, out_hbm, deg_hbm,
           sbuf, dbuf, rows_v, h_sh, acc_sh, *tail):
    ssems = tail[:IB]
    dsems = tail[IB:2 * IB]
    gsems = tail[2 * IB:2 * IB + NB]
    ones_v, deg_sh = tail[2 * IB + NB:]

    c = lax.axis_index("c")
    s = lax.axis_index("s")

    pltpu.sync_copy(h_hbm.at[c, pl.ds(s * RPS, RPS)],
                    h_sh.at[pl.ds(s * RPS, RPS)])
    _zero_acc(rows_v, acc_sh, s)
    for i in range(CH // 16):
      ones_v[pl.ds(i * 16, 16)] = jnp.zeros((16,), jnp.float32)
    for t in range(RPS // CH):
      pltpu.sync_copy(ones_v, deg_sh.at[pl.ds(s * RPS + t * CH, CH)])
    for i in range(CH // 16):
      ones_v[pl.ds(i * 16, 16)] = jnp.ones((16,), jnp.float32)
    plsc.subcore_barrier()

    def deg_chunk(j, q):
      @pl.when((j < NCHUNK1 // 2) == (c == 0))
      def _():
        pltpu.sync_copy(ones_v, deg_sh.at[dbuf.at[q]], add=True)

    _edge_loop(NCHUNK1, src_hbm, dst_hbm, lambda j: (s, j), sbuf, dbuf,
               rows_v, h_sh, acc_sh, ssems, dsems, gsems, deg_chunk)
    plsc.subcore_barrier()

    pltpu.sync_copy(acc_sh.at[pl.ds(s * RPS, RPS)],
                    out_hbm.at[c, pl.ds(s * RPS, RPS)])
    pltpu.sync_copy(deg_sh.at[pl.ds(s * RPS, RPS)],
                    deg_hbm.at[c, pl.ds(s * RPS, RPS)])

  return pl.kernel(body, out_type=out_type, mesh=mesh,
                   scratch_types=_sc_scratch(True),
                   compiler_params=pltpu.CompilerParams(
                       use_tc_tiling_on_sc=False),
                   name="sc_layer_deg")


def _make_sc_layer_nodeg():
  mesh = plsc.VectorSubcoreMesh(
      core_axis_name="c", subcore_axis_name="s", num_cores=NC, num_subcores=NS)

  out_type = jax.ShapeDtypeStruct((NC, NP, DC), jnp.float32)

  def body(h_hbm, src_hbm, dst_hbm, out_hbm,
           sbuf, dbuf, rows_v, h_sh, acc_sh, *tail):
    ssems = tail[:IB]
    dsems = tail[IB:2 * IB]
    gsems = tail[2 * IB:2 * IB + NB]

    c = lax.axis_index("c")
    s = lax.axis_index("s")

    pltpu.sync_copy(h_hbm.at[c, pl.ds(s * RPS, RPS)],
                    h_sh.at[pl.ds(s * RPS, RPS)])
    _zero_acc(rows_v, acc_sh, s)
    plsc.subcore_barrier()

    _edge_loop(NCHUNK1, src_hbm, dst_hbm, lambda j: (s, j), sbuf, dbuf,
               rows_v, h_sh, acc_sh, ssems, dsems, gsems)
    plsc.subcore_barrier()

    pltpu.sync_copy(acc_sh.at[pl.ds(s * RPS, RPS)],
                    out_hbm.at[c, pl.ds(s * RPS, RPS)])

  return pl.kernel(body, out_type=out_type, mesh=mesh,
                   scratch_types=_sc_scratch(False),
                   compiler_params=pltpu.CompilerParams(
                       use_tc_tiling_on_sc=False),
                   name="sc_layer")


def _make_sc_split():
  mesh = plsc.VectorSubcoreMesh(
      core_axis_name="c", subcore_axis_name="s", num_cores=NC, num_subcores=NS)

  out_type = jax.ShapeDtypeStruct((NC, NP, DC), jnp.float32)

  def body(h_hbm, src_hbm, dst_hbm, out_hbm,
           sbuf, dbuf, rows_v, h_sh, acc_sh, *tail):
    ssems = tail[:IB]
    dsems = tail[IB:2 * IB]
    gsems = tail[2 * IB:2 * IB + NB]

    c = lax.axis_index("c")
    s = lax.axis_index("s")

    pltpu.sync_copy(h_hbm.at[pl.ds(s * RPS, RPS)],
                    h_sh.at[pl.ds(s * RPS, RPS)])
    _zero_acc(rows_v, acc_sh, s)
    plsc.subcore_barrier()

    _edge_loop(NCHUNK2, src_hbm, dst_hbm, lambda j: (s, c * NCHUNK2 + j),
               sbuf, dbuf, rows_v, h_sh, acc_sh, ssems, dsems, gsems)
    plsc.subcore_barrier()

    pltpu.sync_copy(acc_sh.at[pl.ds(s * RPS, RPS)],
                    out_hbm.at[c, pl.ds(s * RPS, RPS)])

  return pl.kernel(body, out_type=out_type, mesh=mesh,
                   scratch_types=_sc_scratch(False),
                   compiler_params=pltpu.CompilerParams(
                       use_tc_tiling_on_sc=False),
                   name="sc_split")


_sc_layer_deg = _make_sc_layer()
_sc_layer = _make_sc_layer_nodeg()
_sc_split = _make_sc_split()


def _dot(a, b):
  return jnp.dot(a.astype(jnp.bfloat16), b.astype(jnp.bfloat16),
                 preferred_element_type=jnp.float32)


def _tc_layer0_body(h_ref, a_ref, d_ref, ws_ref, wn_ref, b_ref,
                    o_ref, invd_ref):
  invd = 1.0 / jnp.maximum(d_ref[0] + d_ref[1], 1.0)
  invd_ref[...] = invd
  agg = jnp.concatenate([a_ref[0] * invd, a_ref[1] * invd], axis=1)
  y = _dot(h_ref[...], ws_ref[...]) + _dot(agg, wn_ref[...]) + b_ref[...]
  h1 = jnp.maximum(y, 0.0)
  o_ref[0] = h1[:, :DC]
  o_ref[1] = h1[:, DC:]


def _tc_layer1_body(h_ref, a_ref, invd_ref, ws_ref, wn_ref, b_ref,
                    wn2_ref, o_ref, z_ref):
  agg = jnp.concatenate(
      [a_ref[0] * invd_ref[...], a_ref[1] * invd_ref[...]], axis=1)
  h = jnp.concatenate([h_ref[0], h_ref[1]], axis=1)
  y = _dot(h, ws_ref[...]) + _dot(agg, wn_ref[...]) + b_ref[...]
  h2 = jnp.maximum(y, 0.0)
  o_ref[...] = h2
  z_ref[...] = _dot(h2, wn2_ref[...])


def _tc_final_body(h_ref, p_ref, invd_ref, ws_ref, b_ref, o_ref):
  agg = (p_ref[0] + p_ref[1]) * invd_ref[...]
  o_ref[...] = _dot(h_ref[...], ws_ref[...]) + agg + b_ref[...]


def _row_block(d):
  return pl.BlockSpec((R, d), lambda i: (i, 0))


def _half_block():
  return pl.BlockSpec((2, R, DC), lambda i: (0, i, 0))


def _part_block(d):
  return pl.BlockSpec((NC, R, d), lambda i: (0, i, 0))


def _full(shape):
  return pl.BlockSpec(shape, lambda i: tuple(0 for _ in shape))


_tc_layer0 = pl.pallas_call(
    _tc_layer0_body,
    grid=(GRID,),
    in_specs=[_row_block(D_H), _part_block(DC), _part_block(1),
              _full((D_IN, D_H)), _full((D_IN, D_H)), _full((1, D_H))],
    out_specs=[_half_block(), _row_block(1)],
    out_shape=[jax.ShapeDtypeStruct((2, NP, DC), jnp.float32),
               jax.ShapeDtypeStruct((NP, 1), jnp.float32)],
)

_tc_layer1 = pl.pallas_call(
    _tc_layer1_body,
    grid=(GRID,),
    in_specs=[_half_block(), _part_block(DC), _row_block(1),
              _full((D_H, D_H)), _full((D_H, D_H)), _full((1, D_H)),
              _full((D_H, D_OUT))],
    out_specs=[_row_block(D_H), _row_block(D_OUT)],
    out_shape=[jax.ShapeDtypeStruct((NP, D_H), jnp.float32),
               jax.ShapeDtypeStruct((NP, D_OUT), jnp.float32)],
)

_tc_final = pl.pallas_call(
    _tc_final_body,
    grid=(GRID,),
    in_specs=[_row_block(D_H), _part_block(D_OUT), _row_block(1),
              _full((D_H, D_OUT)), _full((1, D_OUT))],
    out_specs=_row_block(D_OUT),
    out_shape=jax.ShapeDtypeStruct((NP, D_OUT), jnp.float32),
)


def kernel(x, edge_index, W_self0, W_neigh0, b0, W_self1, W_neigh1, b1,
           W_self2, W_neigh2, b2):
  src = edge_index[0]
  dst = edge_index[1]
  src1 = jnp.concatenate(
      [src, jnp.zeros((EPAD - E,), jnp.int32)]).reshape(NS, NCHUNK1, CH)
  dst1 = jnp.concatenate(
      [dst, jnp.full((EPAD - E,), N, jnp.int32)]).reshape(NS, NCHUNK1, CH)

  h0 = jnp.pad(x, ((0, NP - N), (0, 0)))
  h0h = jnp.stack([h0[:, :DC], h0[:, DC:]])

  a0, degp = _sc_layer_deg(h0h, src1, dst1)
  h1h, invd = _tc_layer0(h0, a0, degp[..., None], W_self0, W_neigh0,
                         b0.reshape(1, D_H))
  a1 = _sc_layer(h1h, src1, dst1)
  h2, z2 = _tc_layer1(h1h, a1, invd, W_self1, W_neigh1,
                      b1.reshape(1, D_H), W_neigh2)
  pz = _sc_split(z2, src1, dst1)
  out = _tc_final(h2, pz, invd, W_self2, b2.reshape(1, D_OUT))
  return out[:N]


# speedup vs baseline: 2.9123x; 1.1539x over previous
import jax
import jax.numpy as jnp
from jax import lax
from jax.experimental import pallas as pl
from jax.experimental.pallas import tpu as pltpu
from jax.experimental.pallas import tpu_sc as plsc

N = 10000
E = 320000
D_IN = 128
D_H = 128
D_OUT = 64
DC = 64

NC = 2
NS = 16
NW = NC * NS

NP = 10240
RPS = NP // NS
CH = 128
NB = 2
IB = 4

EW1 = ((E // NS + IB * CH - 1) // (IB * CH)) * (IB * CH)
NCHUNK1 = EW1 // CH
NCHUNK2 = NCHUNK1 // 2
EPAD = NS * EW1

R = 1280
GRID = NP // R


def _sc_scratch(with_deg):
  scratch = [
      pltpu.VMEM((IB, CH), jnp.int32),
      pltpu.VMEM((IB, CH), jnp.int32),
      pltpu.VMEM((NB, CH, DC), jnp.float32),
      pltpu.VMEM_SHARED((NP, DC), jnp.float32),
      pltpu.VMEM_SHARED((NP, DC), jnp.float32),
  ] + [pltpu.SemaphoreType.DMA] * (2 * IB + NB)
  if with_deg:
    scratch += [
        pltpu.VMEM((CH,), jnp.float32),
        pltpu.VMEM_SHARED((NP,), jnp.float32),
    ]
  return scratch


def _zero_acc(rows_v, acc_sh, s):
  for i in range(CH):
    for k in range(DC // 16):
      rows_v[0, i, pl.ds(k * 16, 16)] = jnp.zeros((16,), jnp.float32)
  for t in range(RPS // CH):
    pltpu.sync_copy(rows_v.at[0], acc_sh.at[pl.ds(s * RPS + t * CH, CH)])


def _edge_loop(nchunk, src_hbm, dst_hbm, idx_off, sbuf, dbuf, rows_v, h_sh,
               acc_sh, ssems, dsems, gsems, deg_chunk=None):
  def start_idx(j, q):
    o = idx_off(j)
    pltpu.async_copy(src_hbm.at[pl.ds(o, CH)], sbuf.at[q], ssems[q])
    pltpu.async_copy(dst_hbm.at[pl.ds(o, CH)], dbuf.at[q], dsems[q])

  def wait_idx(j, q):
    o = idx_off(j)
    pltpu.make_async_copy(src_hbm.at[pl.ds(o, CH)], sbuf.at[q],
                          ssems[q]).wait()
    pltpu.make_async_copy(dst_hbm.at[pl.ds(o, CH)], dbuf.at[q],
                          dsems[q]).wait()

  def start_gather(q, b):
    pltpu.async_copy(h_sh.at[sbuf.at[q]], rows_v.at[b], gsems[b])

  def wait_gather(q, b):
    pltpu.make_async_copy(h_sh.at[sbuf.at[q]], rows_v.at[b], gsems[b]).wait()

  for q in range(IB):
    start_idx(q, q)
  for j in range(NB):
    wait_idx(j, j)
    start_gather(j, j)

  def group(g, carry):
    for u in range(IB):
      q = u
      b = u % NB
      j = g * IB + u
      wait_gather(q, b)
      pltpu.sync_copy(rows_v.at[b], acc_sh.at[dbuf.at[q]], add=True)
      if deg_chunk is not None:
        deg_chunk(j, q)

      @pl.when(j + IB < nchunk)
      def _():
        start_idx(j + IB, q)

      @pl.when(j + NB < nchunk)
      def _():
        wait_idx(j + NB, (u + NB) % IB)
        start_gather((u + NB) % IB, b)
    return carry

  lax.fori_loop(0, nchunk // IB, group, 0)


def _make_sc_layer():
  mesh = plsc.VectorSubcoreMesh(
      core_axis_name="c", subcore_axis_name="s", num_cores=NC, num_subcores=NS)

  out_type = [jax.ShapeDtypeStruct((NP, D_H), jnp.float32),
              jax.ShapeDtypeStruct((NC, NP), jnp.float32)]

  def body(h_hbm, src_hbm, dst_hbm, out_hbm, deg_hbm,
           sbuf, dbuf, rows_v, h_sh, acc_sh, *tail):
    ssems = tail[:IB]
    dsems = tail[IB:2 * IB]
    gsems = tail[2 * IB:2 * IB + NB]
    ones_v, deg_sh = tail[2 * IB + NB:]

    c = lax.axis_index("c")
    s = lax.axis_index("s")

    pltpu.sync_copy(h_hbm.at[pl.ds(s * RPS, RPS), pl.ds(c * DC, DC)],
                    h_sh.at[pl.ds(s * RPS, RPS)])
    _zero_acc(rows_v, acc_sh, s)
    for i in range(CH // 16):
      ones_v[pl.ds(i * 16, 16)] = jnp.zeros((16,), jnp.float32)
    for t in range(RPS // CH):
      pltpu.sync_copy(ones_v, deg_sh.at[pl.ds(s * RPS + t * CH, CH)])
    for i in range(CH // 16):
      ones_v[pl.ds(i * 16, 16)] = jnp.ones((16,), jnp.float32)
    plsc.subcore_barrier()

    def deg_chunk(j, q):
      @pl.when((j < NCHUNK1 // 2) == (c == 0))
      def _():
        pltpu.sync_copy(ones_v, deg_sh.at[dbuf.at[q]], add=True)

    _edge_loop(NCHUNK1, src_hbm, dst_hbm, lambda j: s * EW1 + j * CH,
               sbuf, dbuf, rows_v, h_sh, acc_sh, ssems, dsems, gsems,
               deg_chunk)
    plsc.subcore_barrier()

    pltpu.sync_copy(acc_sh.at[pl.ds(s * RPS, RPS)],
                    out_hbm.at[pl.ds(s * RPS, RPS), pl.ds(c * DC, DC)])
    pltpu.sync_copy(deg_sh.at[pl.ds(s * RPS, RPS)],
                    deg_hbm.at[c, pl.ds(s * RPS, RPS)])

  return pl.kernel(body, out_type=out_type, mesh=mesh,
                   scratch_types=_sc_scratch(True),
                   compiler_params=pltpu.CompilerParams(
                       use_tc_tiling_on_sc=False),
                   name="sc_layer_deg")


def _make_sc_layer_nodeg():
  mesh = plsc.VectorSubcoreMesh(
      core_axis_name="c", subcore_axis_name="s", num_cores=NC, num_subcores=NS)

  out_type = jax.ShapeDtypeStruct((NP, D_H), jnp.float32)

  def body(h_hbm, src_hbm, dst_hbm, out_hbm,
           sbuf, dbuf, rows_v, h_sh, acc_sh, *tail):
    ssems = tail[:IB]
    dsems = tail[IB:2 * IB]
    gsems = tail[2 * IB:2 * IB + NB]

    c = lax.axis_index("c")
    s = lax.axis_index("s")

    pltpu.sync_copy(h_hbm.at[pl.ds(s * RPS, RPS), pl.ds(c * DC, DC)],
                    h_sh.at[pl.ds(s * RPS, RPS)])
    _zero_acc(rows_v, acc_sh, s)
    plsc.subcore_barrier()

    _edge_loop(NCHUNK1, src_hbm, dst_hbm, lambda j: s * EW1 + j * CH,
               sbuf, dbuf, rows_v, h_sh, acc_sh, ssems, dsems, gsems)
    plsc.subcore_barrier()

    pltpu.sync_copy(acc_sh.at[pl.ds(s * RPS, RPS)],
                    out_hbm.at[pl.ds(s * RPS, RPS), pl.ds(c * DC, DC)])

  return pl.kernel(body, out_type=out_type, mesh=mesh,
                   scratch_types=_sc_scratch(False),
                   compiler_params=pltpu.CompilerParams(
                       use_tc_tiling_on_sc=False),
                   name="sc_layer")


def _make_sc_split():
  mesh = plsc.VectorSubcoreMesh(
      core_axis_name="c", subcore_axis_name="s", num_cores=NC, num_subcores=NS)

  out_type = jax.ShapeDtypeStruct((NP, D_H), jnp.float32)

  def body(h_hbm, src_hbm, dst_hbm, out_hbm,
           sbuf, dbuf, rows_v, h_sh, acc_sh, *tail):
    ssems = tail[:IB]
    dsems = tail[IB:2 * IB]
    gsems = tail[2 * IB:2 * IB + NB]

    c = lax.axis_index("c")
    s = lax.axis_index("s")

    pltpu.sync_copy(h_hbm.at[pl.ds(s * RPS, RPS), pl.ds(0, DC)],
                    h_sh.at[pl.ds(s * RPS, RPS)])
    _zero_acc(rows_v, acc_sh, s)
    plsc.subcore_barrier()

    _edge_loop(NCHUNK2, src_hbm, dst_hbm,
               lambda j: s * EW1 + (c * NCHUNK2 + j) * CH,
               sbuf, dbuf, rows_v, h_sh, acc_sh, ssems, dsems, gsems)
    plsc.subcore_barrier()

    pltpu.sync_copy(acc_sh.at[pl.ds(s * RPS, RPS)],
                    out_hbm.at[pl.ds(s * RPS, RPS), pl.ds(c * DC, DC)])

  return pl.kernel(body, out_type=out_type, mesh=mesh,
                   scratch_types=_sc_scratch(False),
                   compiler_params=pltpu.CompilerParams(
                       use_tc_tiling_on_sc=False),
                   name="sc_split")


_sc_layer_deg = _make_sc_layer()
_sc_layer = _make_sc_layer_nodeg()
_sc_split = _make_sc_split()


def _dot(a, b):
  return jnp.dot(a.astype(jnp.bfloat16), b.astype(jnp.bfloat16),
                 preferred_element_type=jnp.float32)


def _tc_layer0_body(h_ref, a_ref, d_ref, ws_ref, wn_ref, b_ref,
                    o_ref, invd_ref):
  invd = 1.0 / jnp.maximum(d_ref[0] + d_ref[1], 1.0)
  invd_col = jnp.transpose(invd[None, :])
  invd_bc = jnp.broadcast_to(invd_col, (R, D_H))
  invd_ref[...] = invd_bc
  y = (_dot(h_ref[...], ws_ref[...])
       + _dot(a_ref[...] * invd_bc, wn_ref[...]) + b_ref[...])
  o_ref[...] = jnp.maximum(y, 0.0)


def _tc_layer1_body(h_ref, a_ref, invd_ref, ws_ref, wn_ref, b_ref,
                    wn2_ref, o_ref, z_ref):
  y = (_dot(h_ref[...], ws_ref[...])
       + _dot(a_ref[...] * invd_ref[...], wn_ref[...]) + b_ref[...])
  h2 = jnp.maximum(y, 0.0)
  o_ref[...] = h2
  z = _dot(h2, wn2_ref[...])
  z_ref[...] = jnp.concatenate([z, jnp.zeros((R, D_H - D_OUT),
                                             jnp.float32)], axis=1)


def _tc_final_body(h_ref, p_ref, invd_ref, ws_ref, b_ref, o_ref):
  agg = (p_ref[:, :D_OUT] + p_ref[:, D_OUT:]) * invd_ref[:, :D_OUT]
  o_ref[...] = _dot(h_ref[...], ws_ref[...]) + agg + b_ref[...]


def _row_block(d):
  return pl.BlockSpec((R, d), lambda i: (i, 0))


def _full(shape):
  return pl.BlockSpec(shape, lambda i: tuple(0 for _ in shape))


_tc_layer0 = pl.pallas_call(
    _tc_layer0_body,
    grid=(GRID,),
    in_specs=[_row_block(D_H), _row_block(D_H),
              pl.BlockSpec((NC, R), lambda i: (0, i)),
              _full((D_IN, D_H)), _full((D_IN, D_H)), _full((1, D_H))],
    out_specs=[_row_block(D_H), _row_block(D_H)],
    out_shape=[jax.ShapeDtypeStruct((NP, D_H), jnp.float32),
               jax.ShapeDtypeStruct((NP, D_H), jnp.float32)],
)

_tc_layer1 = pl.pallas_call(
    _tc_layer1_body,
    grid=(GRID,),
    in_specs=[_row_block(D_H), _row_block(D_H), _row_block(D_H),
              _full((D_H, D_H)), _full((D_H, D_H)), _full((1, D_H)),
              _full((D_H, D_OUT))],
    out_specs=[_row_block(D_H), _row_block(D_H)],
    out_shape=[jax.ShapeDtypeStruct((NP, D_H), jnp.float32),
               jax.ShapeDtypeStruct((NP, D_H), jnp.float32)],
)

_tc_final = pl.pallas_call(
    _tc_final_body,
    grid=(GRID,),
    in_specs=[_row_block(D_H), _row_block(D_H), _row_block(D_H),
              _full((D_H, D_OUT)), _full((1, D_OUT))],
    out_specs=_row_block(D_OUT),
    out_shape=jax.ShapeDtypeStruct((NP, D_OUT), jnp.float32),
)


def kernel(x, edge_index, W_self0, W_neigh0, b0, W_self1, W_neigh1, b1,
           W_self2, W_neigh2, b2):
  src = jnp.concatenate([edge_index[0], jnp.zeros((EPAD - E,), jnp.int32)])
  dst = jnp.concatenate([edge_index[1], jnp.full((EPAD - E,), N, jnp.int32)])

  h0 = jnp.pad(x, ((0, NP - N), (0, 0)))

  a0, degp = _sc_layer_deg(h0, src, dst)
  h1, invd = _tc_layer0(h0, a0, degp, W_self0, W_neigh0, b0.reshape(1, D_H))
  a1 = _sc_layer(h1, src, dst)
  h2, z2 = _tc_layer1(h1, a1, invd, W_self1, W_neigh1,
                      b1.reshape(1, D_H), W_neigh2)
  pz = _sc_split(z2, src, dst)
  out = _tc_final(h2, pz, invd, W_self2, b2.reshape(1, D_OUT))
  return out[:N]


# speedup vs baseline: 3.0460x; 1.0459x over previous
import jax
import jax.numpy as jnp
from jax import lax
from jax.experimental import pallas as pl
from jax.experimental.pallas import tpu as pltpu
from jax.experimental.pallas import tpu_sc as plsc

N = 10000
E = 320000
D_IN = 128
D_H = 128
D_OUT = 64
DC = 64

NC = 2
NS = 16
NW = NC * NS

NP = 10240
RPS = NP // NS
CH = 128
CB = 128
NB = 2
IB = 4

ESUB = E // NS
NCHUNK1 = ESUB // CH
TAIL = ESUB - NCHUNK1 * CH
C0N = 76

R = 1280
GRID = NP // R


def _sc_scratch(with_deg):
  scratch = [
      pltpu.VMEM((IB, CH), jnp.int32),
      pltpu.VMEM((IB, CH), jnp.int32),
      pltpu.VMEM((NB, CB, DC), jnp.float32),
      pltpu.VMEM_SHARED((NP, DC), jnp.float32),
      pltpu.VMEM_SHARED((NP, DC), jnp.float32),
  ] + [pltpu.SemaphoreType.DMA] * (2 * IB + NB)
  if with_deg:
    scratch += [
        pltpu.VMEM((CB,), jnp.float32),
        pltpu.VMEM_SHARED((NP,), jnp.float32),
    ]
  return scratch


def _zero_acc(rows_v, acc_sh, s):
  for i in range(CB):
    for k in range(DC // 16):
      rows_v[0, i, pl.ds(k * 16, 16)] = jnp.zeros((16,), jnp.float32)
  for t in range(RPS // CB):
    pltpu.sync_copy(rows_v.at[0], acc_sh.at[pl.ds(s * RPS + t * CB, CB)])


def _edge_loop(nchunk, ei_hbm, idx_off, sbuf, dbuf, rows_v, h_sh,
               acc_sh, ssems, dsems, gsems, deg_chunk=None):
  def start_idx(j, q):
    o = idx_off(j)
    pltpu.async_copy(ei_hbm.at[0, pl.ds(o, CH)], sbuf.at[q], ssems[q])
    pltpu.async_copy(ei_hbm.at[1, pl.ds(o, CH)], dbuf.at[q], dsems[q])

  def wait_idx(j, q):
    o = idx_off(j)
    pltpu.make_async_copy(ei_hbm.at[0, pl.ds(o, CH)], sbuf.at[q],
                          ssems[q]).wait()
    pltpu.make_async_copy(ei_hbm.at[1, pl.ds(o, CH)], dbuf.at[q],
                          dsems[q]).wait()

  def start_gather(q, b):
    pltpu.async_copy(h_sh.at[sbuf.at[q]], rows_v.at[b, pl.ds(0, CH)],
                     gsems[b])

  def wait_gather(q, b):
    pltpu.make_async_copy(h_sh.at[sbuf.at[q]], rows_v.at[b, pl.ds(0, CH)],
                          gsems[b]).wait()

  for q in range(IB):
    start_idx(q, q)
  for j in range(NB):
    wait_idx(j, j)
    start_gather(j, j)

  def group(g, carry):
    for u in range(IB):
      q = u
      b = u % NB
      j = g * IB + u
      wait_gather(q, b)
      pltpu.sync_copy(rows_v.at[b, pl.ds(0, CH)], acc_sh.at[dbuf.at[q]],
                      add=True)
      if deg_chunk is not None:
        deg_chunk(j, q)

      @pl.when(j + IB < nchunk)
      def _():
        start_idx(j + IB, q)

      @pl.when(j + NB < nchunk)
      def _():
        wait_idx(j + NB, (u + NB) % IB)
        start_gather((u + NB) % IB, b)
    return carry

  lax.fori_loop(0, nchunk // IB, group, 0)


def _tail_chunk(ei_hbm, off, sbuf, dbuf, rows_v, h_sh, acc_sh,
                ssems, dsems, gsems, deg_tail=None):
  t = pl.ds(0, TAIL)
  pltpu.async_copy(ei_hbm.at[0, pl.ds(off, TAIL)], sbuf.at[0, t], ssems[0])
  pltpu.async_copy(ei_hbm.at[1, pl.ds(off, TAIL)], dbuf.at[0, t], dsems[0])
  pltpu.make_async_copy(ei_hbm.at[0, pl.ds(off, TAIL)], sbuf.at[0, t],
                        ssems[0]).wait()
  pltpu.make_async_copy(ei_hbm.at[1, pl.ds(off, TAIL)], dbuf.at[0, t],
                        dsems[0]).wait()
  pltpu.async_copy(h_sh.at[sbuf.at[0, t]], rows_v.at[0, t], gsems[0])
  pltpu.make_async_copy(h_sh.at[sbuf.at[0, t]], rows_v.at[0, t],
                        gsems[0]).wait()
  pltpu.sync_copy(rows_v.at[0, t], acc_sh.at[dbuf.at[0, t]], add=True)
  if deg_tail is not None:
    deg_tail()


def _make_sc_layer():
  mesh = plsc.VectorSubcoreMesh(
      core_axis_name="c", subcore_axis_name="s", num_cores=NC, num_subcores=NS)

  out_type = [jax.ShapeDtypeStruct((NP, D_H), jnp.float32),
              jax.ShapeDtypeStruct((NC, NP), jnp.float32)]

  def body(h_hbm, ei_hbm, out_hbm, deg_hbm,
           sbuf, dbuf, rows_v, h_sh, acc_sh, *tail):
    ssems = tail[:IB]
    dsems = tail[IB:2 * IB]
    gsems = tail[2 * IB:2 * IB + NB]
    ones_v, deg_sh = tail[2 * IB + NB:]

    c = lax.axis_index("c")
    s = lax.axis_index("s")

    pltpu.sync_copy(h_hbm.at[pl.ds(s * RPS, RPS), pl.ds(c * DC, DC)],
                    h_sh.at[pl.ds(s * RPS, RPS)])
    _zero_acc(rows_v, acc_sh, s)
    for i in range(CB // 16):
      ones_v[pl.ds(i * 16, 16)] = jnp.zeros((16,), jnp.float32)
    for t in range(RPS // CB):
      pltpu.sync_copy(ones_v, deg_sh.at[pl.ds(s * RPS + t * CB, CB)])
    for i in range(CB // 16):
      ones_v[pl.ds(i * 16, 16)] = jnp.ones((16,), jnp.float32)
    plsc.subcore_barrier()

    def deg_chunk(j, q):
      @pl.when((j < NCHUNK1 // 2) == (c == 0))
      def _():
        pltpu.sync_copy(ones_v.at[pl.ds(0, CH)], deg_sh.at[dbuf.at[q]],
                        add=True)

    _edge_loop(NCHUNK1, ei_hbm, lambda j: s * ESUB + j * CH,
               sbuf, dbuf, rows_v, h_sh, acc_sh, ssems, dsems, gsems,
               deg_chunk)

    def deg_tail():
      @pl.when(c == 1)
      def _():
        pltpu.sync_copy(ones_v.at[pl.ds(0, TAIL)],
                        deg_sh.at[dbuf.at[0, pl.ds(0, TAIL)]], add=True)

    _tail_chunk(ei_hbm, s * ESUB + NCHUNK1 * CH, sbuf, dbuf, rows_v,
                h_sh, acc_sh, ssems, dsems, gsems, deg_tail)
    plsc.subcore_barrier()

    pltpu.sync_copy(acc_sh.at[pl.ds(s * RPS, RPS)],
                    out_hbm.at[pl.ds(s * RPS, RPS), pl.ds(c * DC, DC)])
    pltpu.sync_copy(deg_sh.at[pl.ds(s * RPS, RPS)],
                    deg_hbm.at[c, pl.ds(s * RPS, RPS)])

  return pl.kernel(body, out_type=out_type, mesh=mesh,
                   scratch_types=_sc_scratch(True),
                   compiler_params=pltpu.CompilerParams(
                       use_tc_tiling_on_sc=False),
                   name="sc_layer_deg")


def _make_sc_layer_nodeg():
  mesh = plsc.VectorSubcoreMesh(
      core_axis_name="c", subcore_axis_name="s", num_cores=NC, num_subcores=NS)

  out_type = jax.ShapeDtypeStruct((NP, D_H), jnp.float32)

  def body(h_hbm, ei_hbm, out_hbm,
           sbuf, dbuf, rows_v, h_sh, acc_sh, *tail):
    ssems = tail[:IB]
    dsems = tail[IB:2 * IB]
    gsems = tail[2 * IB:2 * IB + NB]

    c = lax.axis_index("c")
    s = lax.axis_index("s")

    pltpu.sync_copy(h_hbm.at[pl.ds(s * RPS, RPS), pl.ds(c * DC, DC)],
                    h_sh.at[pl.ds(s * RPS, RPS)])
    _zero_acc(rows_v, acc_sh, s)
    plsc.subcore_barrier()

    _edge_loop(NCHUNK1, ei_hbm, lambda j: s * ESUB + j * CH,
               sbuf, dbuf, rows_v, h_sh, acc_sh, ssems, dsems, gsems)
    _tail_chunk(ei_hbm, s * ESUB + NCHUNK1 * CH, sbuf, dbuf, rows_v,
                h_sh, acc_sh, ssems, dsems, gsems)
    plsc.subcore_barrier()

    pltpu.sync_copy(acc_sh.at[pl.ds(s * RPS, RPS)],
                    out_hbm.at[pl.ds(s * RPS, RPS), pl.ds(c * DC, DC)])

  return pl.kernel(body, out_type=out_type, mesh=mesh,
                   scratch_types=_sc_scratch(False),
                   compiler_params=pltpu.CompilerParams(
                       use_tc_tiling_on_sc=False),
                   name="sc_layer")


def _make_sc_split():
  mesh = plsc.VectorSubcoreMesh(
      core_axis_name="c", subcore_axis_name="s", num_cores=NC, num_subcores=NS)

  out_type = jax.ShapeDtypeStruct((NP, D_H), jnp.float32)

  def body(h_hbm, ei_hbm, out_hbm,
           sbuf, dbuf, rows_v, h_sh, acc_sh, *tail):
    ssems = tail[:IB]
    dsems = tail[IB:2 * IB]
    gsems = tail[2 * IB:2 * IB + NB]

    c = lax.axis_index("c")
    s = lax.axis_index("s")

    pltpu.sync_copy(h_hbm.at[pl.ds(s * RPS, RPS), pl.ds(0, DC)],
                    h_sh.at[pl.ds(s * RPS, RPS)])
    _zero_acc(rows_v, acc_sh, s)
    plsc.subcore_barrier()

    nchunk = jnp.where(c == 0, C0N, NCHUNK1 - C0N)
    base = jnp.where(c == 0, 0, C0N)
    _edge_loop(nchunk, ei_hbm,
               lambda j: s * ESUB + (base + j) * CH,
               sbuf, dbuf, rows_v, h_sh, acc_sh, ssems, dsems, gsems)

    @pl.when(c == 0)
    def _():
      _tail_chunk(ei_hbm, s * ESUB + NCHUNK1 * CH, sbuf, dbuf, rows_v,
                  h_sh, acc_sh, ssems, dsems, gsems)
    plsc.subcore_barrier()

    pltpu.sync_copy(acc_sh.at[pl.ds(s * RPS, RPS)],
                    out_hbm.at[pl.ds(s * RPS, RPS), pl.ds(c * DC, DC)])

  return pl.kernel(body, out_type=out_type, mesh=mesh,
                   scratch_types=_sc_scratch(False),
                   compiler_params=pltpu.CompilerParams(
                       use_tc_tiling_on_sc=False),
                   name="sc_split")


_sc_layer_deg = _make_sc_layer()
_sc_layer = _make_sc_layer_nodeg()
_sc_split = _make_sc_split()


def _dot(a, b):
  return jnp.dot(a.astype(jnp.bfloat16), b.astype(jnp.bfloat16),
                 preferred_element_type=jnp.float32)


def _tc_layer0_body(h_ref, a_ref, d_ref, ws_ref, wn_ref, b_ref,
                    o_ref, invd_ref):
  invd = 1.0 / jnp.maximum(d_ref[0] + d_ref[1], 1.0)
  invd_col = jnp.transpose(invd[None, :])
  invd_bc = jnp.broadcast_to(invd_col, (R, D_H))
  invd_ref[...] = invd_bc
  y = (_dot(h_ref[...], ws_ref[...])
       + _dot(a_ref[...] * invd_bc, wn_ref[...]) + b_ref[...])
  o_ref[...] = jnp.maximum(y, 0.0)


def _tc_layer1_body(h_ref, a_ref, invd_ref, ws_ref, wn_ref, b_ref,
                    wn2_ref, o_ref, z_ref):
  y = (_dot(h_ref[...], ws_ref[...])
       + _dot(a_ref[...] * invd_ref[...], wn_ref[...]) + b_ref[...])
  h2 = jnp.maximum(y, 0.0)
  o_ref[...] = h2
  z = _dot(h2, wn2_ref[...])
  z_ref[...] = jnp.concatenate([z, jnp.zeros((R, D_H - D_OUT),
                                             jnp.float32)], axis=1)


def _tc_final_body(h_ref, p_ref, invd_ref, ws_ref, b_ref, o_ref):
  agg = (p_ref[:, :D_OUT] + p_ref[:, D_OUT:]) * invd_ref[:, :D_OUT]
  o_ref[...] = _dot(h_ref[...], ws_ref[...]) + agg + b_ref[...]


def _row_block(d):
  return pl.BlockSpec((R, d), lambda i: (i, 0))


def _full(shape):
  return pl.BlockSpec(shape, lambda i: tuple(0 for _ in shape))


_tc_layer0 = pl.pallas_call(
    _tc_layer0_body,
    grid=(GRID,),
    in_specs=[_row_block(D_H), _row_block(D_H),
              pl.BlockSpec((NC, R), lambda i: (0, i)),
              _full((D_IN, D_H)), _full((D_IN, D_H)), _full((1, D_H))],
    out_specs=[_row_block(D_H), _row_block(D_H)],
    out_shape=[jax.ShapeDtypeStruct((NP, D_H), jnp.float32),
               jax.ShapeDtypeStruct((NP, D_H), jnp.float32)],
)

_tc_layer1 = pl.pallas_call(
    _tc_layer1_body,
    grid=(GRID,),
    in_specs=[_row_block(D_H), _row_block(D_H), _row_block(D_H),
              _full((D_H, D_H)), _full((D_H, D_H)), _full((1, D_H)),
              _full((D_H, D_OUT))],
    out_specs=[_row_block(D_H), _row_block(D_H)],
    out_shape=[jax.ShapeDtypeStruct((NP, D_H), jnp.float32),
               jax.ShapeDtypeStruct((NP, D_H), jnp.float32)],
)

_tc_final = pl.pallas_call(
    _tc_final_body,
    grid=(GRID,),
    in_specs=[_row_block(D_H), _row_block(D_H), _row_block(D_H),
              _full((D_H, D_OUT)), _full((1, D_OUT))],
    out_specs=_row_block(D_OUT),
    out_shape=jax.ShapeDtypeStruct((NP, D_OUT), jnp.float32),
)


def kernel(x, edge_index, W_self0, W_neigh0, b0, W_self1, W_neigh1, b1,
           W_self2, W_neigh2, b2):
  h0 = jnp.pad(x, ((0, NP - N), (0, 0)))

  a0, degp = _sc_layer_deg(h0, edge_index)
  h1, invd = _tc_layer0(h0, a0, degp, W_self0, W_neigh0, b0.reshape(1, D_H))
  a1 = _sc_layer(h1, edge_index)
  h2, z2 = _tc_layer1(h1, a1, invd, W_self1, W_neigh1,
                      b1.reshape(1, D_H), W_neigh2)
  pz = _sc_split(z2, edge_index)
  out = _tc_final(h2, pz, invd, W_self2, b2.reshape(1, D_OUT))
  return out[:N]


# speedup vs baseline: 3.0504x; 1.0014x over previous
import jax
import jax.numpy as jnp
from jax import lax
from jax.experimental import pallas as pl
from jax.experimental.pallas import tpu as pltpu
from jax.experimental.pallas import tpu_sc as plsc

N = 10000
E = 320000
D_IN = 128
D_H = 128
D_OUT = 64
DC = 64

NC = 2
NS = 16
NW = NC * NS

NP = 10240
RPS = NP // NS
CH = 128
CB = 128
NB = 2
IB = 4

ESUB = E // NS
NCHUNK1 = ESUB // CH
TAIL = ESUB - NCHUNK1 * CH
C0N = 76

R = 1280
GRID = NP // R


def _sc_scratch(with_deg):
  scratch = [
      pltpu.VMEM((IB, CH), jnp.int32),
      pltpu.VMEM((IB, CH), jnp.int32),
      pltpu.VMEM((NB, CB, DC), jnp.float32),
      pltpu.VMEM_SHARED((NP, DC), jnp.float32),
      pltpu.VMEM_SHARED((NP, DC), jnp.float32),
  ] + [pltpu.SemaphoreType.DMA] * (2 * IB + NB)
  if with_deg:
    scratch += [
        pltpu.VMEM((CB,), jnp.float32),
        pltpu.VMEM_SHARED((NP,), jnp.float32),
    ]
  return scratch


def _zero_acc(rows_v, acc_sh, s):
  for i in range(CB):
    for k in range(DC // 16):
      rows_v[0, i, pl.ds(k * 16, 16)] = jnp.zeros((16,), jnp.float32)
  for t in range(RPS // CB):
    pltpu.sync_copy(rows_v.at[0], acc_sh.at[pl.ds(s * RPS + t * CB, CB)])


def _edge_loop(nchunk, ei_hbm, idx_off, sbuf, dbuf, rows_v, h_sh,
               acc_sh, ssems, dsems, gsems, deg_chunk=None):
  def start_idx(j, q):
    o = idx_off(j)
    pltpu.async_copy(ei_hbm.at[0, pl.ds(o, CH)], sbuf.at[q], ssems[q])
    pltpu.async_copy(ei_hbm.at[1, pl.ds(o, CH)], dbuf.at[q], dsems[q])

  def wait_idx(j, q):
    o = idx_off(j)
    pltpu.make_async_copy(ei_hbm.at[0, pl.ds(o, CH)], sbuf.at[q],
                          ssems[q]).wait()
    pltpu.make_async_copy(ei_hbm.at[1, pl.ds(o, CH)], dbuf.at[q],
                          dsems[q]).wait()

  def start_gather(q, b):
    pltpu.async_copy(h_sh.at[sbuf.at[q]], rows_v.at[b, pl.ds(0, CH)],
                     gsems[b])

  def wait_gather(q, b):
    pltpu.make_async_copy(h_sh.at[sbuf.at[q]], rows_v.at[b, pl.ds(0, CH)],
                          gsems[b]).wait()

  for q in range(IB):
    start_idx(q, q)
  for j in range(NB):
    wait_idx(j, j)
    start_gather(j, j)

  def group(g, carry):
    for u in range(IB):
      q = u
      b = u % NB
      j = g * IB + u
      wait_gather(q, b)
      pltpu.sync_copy(rows_v.at[b, pl.ds(0, CH)], acc_sh.at[dbuf.at[q]],
                      add=True)
      if deg_chunk is not None:
        deg_chunk(j, q)

      @pl.when(j + IB < nchunk)
      def _():
        start_idx(j + IB, q)

      @pl.when(j + NB < nchunk)
      def _():
        wait_idx(j + NB, (u + NB) % IB)
        start_gather((u + NB) % IB, b)
    return carry

  lax.fori_loop(0, nchunk // IB, group, 0)


def _tail_chunk(ei_hbm, off, sbuf, dbuf, rows_v, h_sh, acc_sh,
                ssems, dsems, gsems, deg_tail=None):
  t = pl.ds(0, TAIL)
  pltpu.async_copy(ei_hbm.at[0, pl.ds(off, TAIL)], sbuf.at[0, t], ssems[0])
  pltpu.async_copy(ei_hbm.at[1, pl.ds(off, TAIL)], dbuf.at[0, t], dsems[0])
  pltpu.make_async_copy(ei_hbm.at[0, pl.ds(off, TAIL)], sbuf.at[0, t],
                        ssems[0]).wait()
  pltpu.make_async_copy(ei_hbm.at[1, pl.ds(off, TAIL)], dbuf.at[0, t],
                        dsems[0]).wait()
  pltpu.async_copy(h_sh.at[sbuf.at[0, t]], rows_v.at[0, t], gsems[0])
  pltpu.make_async_copy(h_sh.at[sbuf.at[0, t]], rows_v.at[0, t],
                        gsems[0]).wait()
  pltpu.sync_copy(rows_v.at[0, t], acc_sh.at[dbuf.at[0, t]], add=True)
  if deg_tail is not None:
    deg_tail()


def _make_sc_layer():
  mesh = plsc.VectorSubcoreMesh(
      core_axis_name="c", subcore_axis_name="s", num_cores=NC, num_subcores=NS)

  out_type = [jax.ShapeDtypeStruct((NP, D_H), jnp.float32),
              jax.ShapeDtypeStruct((NC, NP), jnp.float32)]

  def body(h_hbm, ei_hbm, out_hbm, deg_hbm,
           sbuf, dbuf, rows_v, h_sh, acc_sh, *tail):
    ssems = tail[:IB]
    dsems = tail[IB:2 * IB]
    gsems = tail[2 * IB:2 * IB + NB]
    ones_v, deg_sh = tail[2 * IB + NB:]

    c = lax.axis_index("c")
    s = lax.axis_index("s")

    pltpu.sync_copy(h_hbm.at[pl.ds(s * RPS, RPS), pl.ds(c * DC, DC)],
                    h_sh.at[pl.ds(s * RPS, RPS)])
    _zero_acc(rows_v, acc_sh, s)
    for i in range(CB // 16):
      ones_v[pl.ds(i * 16, 16)] = jnp.zeros((16,), jnp.float32)
    for t in range(RPS // CB):
      pltpu.sync_copy(ones_v, deg_sh.at[pl.ds(s * RPS + t * CB, CB)])
    for i in range(CB // 16):
      ones_v[pl.ds(i * 16, 16)] = jnp.ones((16,), jnp.float32)
    plsc.subcore_barrier()

    def deg_chunk(j, q):
      @pl.when((j < NCHUNK1 // 2) == (c == 0))
      def _():
        pltpu.sync_copy(ones_v.at[pl.ds(0, CH)], deg_sh.at[dbuf.at[q]],
                        add=True)

    _edge_loop(NCHUNK1, ei_hbm, lambda j: s * ESUB + j * CH,
               sbuf, dbuf, rows_v, h_sh, acc_sh, ssems, dsems, gsems,
               deg_chunk)

    def deg_tail():
      @pl.when(c == 1)
      def _():
        pltpu.sync_copy(ones_v.at[pl.ds(0, TAIL)],
                        deg_sh.at[dbuf.at[0, pl.ds(0, TAIL)]], add=True)

    _tail_chunk(ei_hbm, s * ESUB + NCHUNK1 * CH, sbuf, dbuf, rows_v,
                h_sh, acc_sh, ssems, dsems, gsems, deg_tail)
    plsc.subcore_barrier()

    pltpu.sync_copy(acc_sh.at[pl.ds(s * RPS, RPS)],
                    out_hbm.at[pl.ds(s * RPS, RPS), pl.ds(c * DC, DC)])
    pltpu.sync_copy(deg_sh.at[pl.ds(s * RPS, RPS)],
                    deg_hbm.at[c, pl.ds(s * RPS, RPS)])

  return pl.kernel(body, out_type=out_type, mesh=mesh,
                   scratch_types=_sc_scratch(True),
                   compiler_params=pltpu.CompilerParams(
                       use_tc_tiling_on_sc=False),
                   name="sc_layer_deg")


def _make_sc_layer_nodeg():
  mesh = plsc.VectorSubcoreMesh(
      core_axis_name="c", subcore_axis_name="s", num_cores=NC, num_subcores=NS)

  out_type = jax.ShapeDtypeStruct((NP, D_H), jnp.float32)

  def body(h_hbm, ei_hbm, out_hbm,
           sbuf, dbuf, rows_v, h_sh, acc_sh, *tail):
    ssems = tail[:IB]
    dsems = tail[IB:2 * IB]
    gsems = tail[2 * IB:2 * IB + NB]

    c = lax.axis_index("c")
    s = lax.axis_index("s")

    pltpu.sync_copy(h_hbm.at[pl.ds(s * RPS, RPS), pl.ds(c * DC, DC)],
                    h_sh.at[pl.ds(s * RPS, RPS)])
    _zero_acc(rows_v, acc_sh, s)
    plsc.subcore_barrier()

    _edge_loop(NCHUNK1, ei_hbm, lambda j: s * ESUB + j * CH,
               sbuf, dbuf, rows_v, h_sh, acc_sh, ssems, dsems, gsems)
    _tail_chunk(ei_hbm, s * ESUB + NCHUNK1 * CH, sbuf, dbuf, rows_v,
                h_sh, acc_sh, ssems, dsems, gsems)
    plsc.subcore_barrier()

    pltpu.sync_copy(acc_sh.at[pl.ds(s * RPS, RPS)],
                    out_hbm.at[pl.ds(s * RPS, RPS), pl.ds(c * DC, DC)])

  return pl.kernel(body, out_type=out_type, mesh=mesh,
                   scratch_types=_sc_scratch(False),
                   compiler_params=pltpu.CompilerParams(
                       use_tc_tiling_on_sc=False),
                   name="sc_layer")


def _make_sc_split():
  mesh = plsc.VectorSubcoreMesh(
      core_axis_name="c", subcore_axis_name="s", num_cores=NC, num_subcores=NS)

  out_type = jax.ShapeDtypeStruct((NP, D_H), jnp.float32)

  def body(h_hbm, ei_hbm, out_hbm,
           sbuf, dbuf, rows_v, h_sh, acc_sh, *tail):
    ssems = tail[:IB]
    dsems = tail[IB:2 * IB]
    gsems = tail[2 * IB:2 * IB + NB]

    c = lax.axis_index("c")
    s = lax.axis_index("s")

    pltpu.sync_copy(h_hbm.at[pl.ds(s * RPS, RPS), pl.ds(0, DC)],
                    h_sh.at[pl.ds(s * RPS, RPS)])
    _zero_acc(rows_v, acc_sh, s)
    plsc.subcore_barrier()

    nchunk = jnp.where(c == 0, C0N, NCHUNK1 - C0N)
    base = jnp.where(c == 0, 0, C0N)
    _edge_loop(nchunk, ei_hbm,
               lambda j: s * ESUB + (base + j) * CH,
               sbuf, dbuf, rows_v, h_sh, acc_sh, ssems, dsems, gsems)

    @pl.when(c == 0)
    def _():
      _tail_chunk(ei_hbm, s * ESUB + NCHUNK1 * CH, sbuf, dbuf, rows_v,
                  h_sh, acc_sh, ssems, dsems, gsems)
    plsc.subcore_barrier()

    pltpu.sync_copy(acc_sh.at[pl.ds(s * RPS, RPS)],
                    out_hbm.at[pl.ds(s * RPS, RPS), pl.ds(c * DC, DC)])

  return pl.kernel(body, out_type=out_type, mesh=mesh,
                   scratch_types=_sc_scratch(False),
                   compiler_params=pltpu.CompilerParams(
                       use_tc_tiling_on_sc=False),
                   name="sc_split")


_sc_layer_deg = _make_sc_layer()
_sc_layer = _make_sc_layer_nodeg()
_sc_split = _make_sc_split()


def _dot(a, b):
  return jnp.dot(a.astype(jnp.bfloat16), b.astype(jnp.bfloat16),
                 preferred_element_type=jnp.float32)


def _invd_col(d_ref):
  invd = 1.0 / jnp.maximum(d_ref[0] + d_ref[1], 1.0)
  return jnp.transpose(invd[None, :])


def _tc_layer0_body(h_ref, a_ref, d_ref, ws_ref, wn_ref, b_ref, o_ref):
  y = (_dot(h_ref[...], ws_ref[...])
       + _dot(a_ref[...] * _invd_col(d_ref), wn_ref[...]) + b_ref[...])
  o_ref[...] = jnp.maximum(y, 0.0)


def _tc_layer1_body(h_ref, a_ref, d_ref, ws_ref, wn_ref, b_ref,
                    wn2_ref, o_ref, z_ref):
  y = (_dot(h_ref[...], ws_ref[...])
       + _dot(a_ref[...] * _invd_col(d_ref), wn_ref[...]) + b_ref[...])
  h2 = jnp.maximum(y, 0.0)
  o_ref[...] = h2
  z = _dot(h2, wn2_ref[...])
  z_ref[...] = jnp.concatenate([z, jnp.zeros((R, D_H - D_OUT),
                                             jnp.float32)], axis=1)


def _tc_final_body(h_ref, p_ref, d_ref, ws_ref, b_ref, o_ref):
  agg = (p_ref[:, :D_OUT] + p_ref[:, D_OUT:]) * _invd_col(d_ref)
  o_ref[...] = _dot(h_ref[...], ws_ref[...]) + agg + b_ref[...]


def _row_block(d):
  return pl.BlockSpec((R, d), lambda i: (i, 0))


def _full(shape):
  return pl.BlockSpec(shape, lambda i: tuple(0 for _ in shape))


_deg_block = pl.BlockSpec((NC, R), lambda i: (0, i))

_tc_layer0 = pl.pallas_call(
    _tc_layer0_body,
    grid=(GRID,),
    in_specs=[_row_block(D_H), _row_block(D_H), _deg_block,
              _full((D_IN, D_H)), _full((D_IN, D_H)), _full((1, D_H))],
    out_specs=_row_block(D_H),
    out_shape=jax.ShapeDtypeStruct((NP, D_H), jnp.float32),
)

_tc_layer1 = pl.pallas_call(
    _tc_layer1_body,
    grid=(GRID,),
    in_specs=[_row_block(D_H), _row_block(D_H), _deg_block,
              _full((D_H, D_H)), _full((D_H, D_H)), _full((1, D_H)),
              _full((D_H, D_OUT))],
    out_specs=[_row_block(D_H), _row_block(D_H)],
    out_shape=[jax.ShapeDtypeStruct((NP, D_H), jnp.float32),
               jax.ShapeDtypeStruct((NP, D_H), jnp.float32)],
)

_tc_final = pl.pallas_call(
    _tc_final_body,
    grid=(GRID,),
    in_specs=[_row_block(D_H), _row_block(D_H), _deg_block,
              _full((D_H, D_OUT)), _full((1, D_OUT))],
    out_specs=_row_block(D_OUT),
    out_shape=jax.ShapeDtypeStruct((N, D_OUT), jnp.float32),
)


def kernel(x, edge_index, W_self0, W_neigh0, b0, W_self1, W_neigh1, b1,
           W_self2, W_neigh2, b2):
  h0 = jnp.pad(x, ((0, NP - N), (0, 0)))

  a0, degp = _sc_layer_deg(h0, edge_index)
  h1 = _tc_layer0(h0, a0, degp, W_self0, W_neigh0, b0.reshape(1, D_H))
  a1 = _sc_layer(h1, edge_index)
  h2, z2 = _tc_layer1(h1, a1, degp, W_self1, W_neigh1,
                      b1.reshape(1, D_H), W_neigh2)
  pz = _sc_split(z2, edge_index)
  return _tc_final(h2, pz, degp, W_self2, b2.reshape(1, D_OUT))
